# baseline (speedup 1.0000x reference)
import functools

import jax
import jax.numpy as jnp
from jax import lax
from jax.experimental import pallas as pl
from jax.experimental.pallas import tpu as pltpu
from jax.experimental.pallas import tpu_sc as plsc

N_NODES = 10000
N_EDGES = 160000
EH = N_EDGES // 2
D = 128
LN_EPS = 1e-6

NC = 2
NS = 16
NW = NC * NS


def _dense_tc(x, W, b, brows, in_off=0, rows=None):
    M, K = x.shape
    rows = rows if rows is not None else M
    Dout = W.shape[1]
    b2 = b.reshape(1, Dout)

    def body(x_ref, w_ref, b_ref, o_ref):
        y = jnp.dot(x_ref[...], w_ref[...],
                    preferred_element_type=jnp.float32) + b_ref[...]
        o_ref[...] = y

    return pl.pallas_call(
        body,
        grid=(rows // brows,),
        in_specs=[
            pl.BlockSpec((brows, K), lambda i: (i + in_off, 0)),
            pl.BlockSpec((K, Dout), lambda i: (0, 0)),
            pl.BlockSpec((1, Dout), lambda i: (0, 0)),
        ],
        out_specs=pl.BlockSpec((brows, Dout), lambda i: (i, 0)),
        out_shape=jax.ShapeDtypeStruct((rows, Dout), jnp.float32),
    )(x, W, b2)


def _prep_tc(g, Weg, be, Wag, ba, Wng, bn):
    def body(g_ref, weg, be_r, wag, ba_r, wng, bn_r, gew, gaw, gnw):
        gv = g_ref[...]
        gew[...] = jnp.dot(gv, weg[...], preferred_element_type=jnp.float32) + be_r[...]
        gaw[...] = jnp.dot(gv, wag[...], preferred_element_type=jnp.float32) + ba_r[...]
        gnw[...] = jnp.dot(gv, wng[...], preferred_element_type=jnp.float32) + bn_r[...]

    out = jax.ShapeDtypeStruct((1, D), jnp.float32)
    return pl.pallas_call(body, out_shape=(out, out, out))(
        g, Weg, be.reshape(1, D), Wag, ba.reshape(1, D), Wng, bn.reshape(1, D))


def _edge_fused_tc(elat, sent, recv, W0, W1, W2, A0, gew, gaw, brows=2000):
    M = elat.shape[0]

    def body(e_ref, s_ref, r_ref, w0, w1, w2, a0, gew_r, gaw_r, ex_ref, p_ref):
        ne = jnp.dot(e_ref[...].astype(jnp.bfloat16), w0[...],
                     preferred_element_type=jnp.float32)
        ne += jnp.dot(s_ref[...].astype(jnp.bfloat16), w1[...],
                      preferred_element_type=jnp.float32)
        ne += jnp.dot(r_ref[...].astype(jnp.bfloat16), w2[...],
                      preferred_element_type=jnp.float32)
        ne = jnp.maximum(ne + gew_r[...], 0.0)
        lg = jnp.dot(ne.astype(jnp.bfloat16), a0[...],
                     preferred_element_type=jnp.float32)
        lg = jnp.maximum(lg + gaw_r[...], 0.0)
        ex = jnp.exp(lg)
        ex_ref[...] = ex
        p_ref[...] = ex * ne

    blk = pl.BlockSpec((brows, D), lambda i: (i, 0))
    wblk = pl.BlockSpec((D, D), lambda i: (0, 0))
    vblk = pl.BlockSpec((1, D), lambda i: (0, 0))
    out = jax.ShapeDtypeStruct((M, D), jnp.float32)
    bf = jnp.bfloat16
    return pl.pallas_call(
        body,
        grid=(M // brows,),
        in_specs=[blk, blk, blk, wblk, wblk, wblk, wblk, vblk, vblk],
        out_specs=(blk, blk),
        out_shape=(out, out),
    )(elat, sent, recv, W0.astype(bf), W1.astype(bf), W2.astype(bf),
      A0.astype(bf), gew, gaw)


def _nd_combine_tc(dq, brows=2048):
    N = dq.shape[1]

    def body(a_ref, ri_ref):
        d = a_ref[0] + a_ref[1]
        ri_ref[...] = 1.0 / jnp.where(d == 0.0, 1.0, d)

    blk3 = pl.BlockSpec((2, brows, D), lambda i: (0, i, 0))
    blk = pl.BlockSpec((brows, D), lambda i: (i, 0))
    return pl.pallas_call(
        body,
        grid=(N // brows,),
        in_specs=[blk3],
        out_specs=blk,
        out_shape=jax.ShapeDtypeStruct((N, D), jnp.float32),
    )(dq)


def _att_fused_tc(p, rinvr, elat, ln_scale, ln_bias, brows=2000):
    M = p.shape[0]

    def body(p_ref, dr_ref, el_ref, sc_ref, bi_ref, att_ref, eo_ref, ea_ref):
        att = p_ref[...] * dr_ref[...]
        att_ref[...] = att
        x = att + el_ref[...]
        mu = jnp.mean(x, axis=-1, keepdims=True)
        xc = x - mu
        var = jnp.mean(xc * xc, axis=-1, keepdims=True)
        eo_ref[...] = xc * lax.rsqrt(var + LN_EPS) * sc_ref[...] + bi_ref[...]

        @pl.when(pl.program_id(0) == 0)
        def _():
            ea_ref[...] = jnp.zeros_like(ea_ref)

        ea_ref[...] += jnp.sum(att, axis=0, keepdims=True)

    blk = pl.BlockSpec((brows, D), lambda i: (i, 0))
    vblk = pl.BlockSpec((1, D), lambda i: (0, 0))
    out = jax.ShapeDtypeStruct((M, D), jnp.float32)
    outv = jax.ShapeDtypeStruct((1, D), jnp.float32)
    return pl.pallas_call(
        body,
        grid=(M // brows,),
        in_specs=[blk, blk, blk, vblk, vblk],
        out_specs=(blk, blk, vblk),
        out_shape=(out, out, outv),
    )(p, rinvr, elat, ln_scale.reshape(1, D), ln_bias.reshape(1, D))


def _node_fused_tc(nlat, spa, spb, Wn0, Wn1, Wn2, gnw, ln_scale,
                   ln_bias, brows=2000):
    N = nlat.shape[0]

    def body(n_ref, spa_ref, spb_ref, w0, w1, w2, gnw_r, sc_ref, bi_ref,
             no_ref, na_ref):
        sa = spa_ref[0] + spb_ref[0]
        rv = spa_ref[1] + spb_ref[1]
        nn = jnp.dot(n_ref[...], w0[...], preferred_element_type=jnp.float32)
        nn += jnp.dot(sa, w1[...], preferred_element_type=jnp.float32)
        nn += jnp.dot(rv, w2[...], preferred_element_type=jnp.float32)
        nn = jnp.maximum(nn + gnw_r[...], 0.0)

        @pl.when(pl.program_id(0) == 0)
        def _():
            na_ref[...] = jnp.zeros_like(na_ref)

        na_ref[...] += jnp.sum(nn, axis=0, keepdims=True)
        x = nn + n_ref[...]
        mu = jnp.mean(x, axis=-1, keepdims=True)
        xc = x - mu
        var = jnp.mean(xc * xc, axis=-1, keepdims=True)
        no_ref[...] = xc * lax.rsqrt(var + LN_EPS) * sc_ref[...] + bi_ref[...]

    blk = pl.BlockSpec((brows, D), lambda i: (i, 0))
    blk3 = pl.BlockSpec((2, brows, D), lambda i: (0, i, 0))
    wblk = pl.BlockSpec((D, D), lambda i: (0, 0))
    vblk = pl.BlockSpec((1, D), lambda i: (0, 0))
    return pl.pallas_call(
        body,
        grid=(N // brows,),
        in_specs=[blk, blk3, blk3, wblk, wblk, wblk, vblk, vblk, vblk],
        out_specs=(blk, vblk),
        out_shape=(jax.ShapeDtypeStruct((N, D), jnp.float32),
                   jax.ShapeDtypeStruct((1, D), jnp.float32)),
    )(nlat, spa, spb, Wn0, Wn1, Wn2, gnw,
      ln_scale.reshape(1, D), ln_bias.reshape(1, D))


def _gup_tc(node_attr, ea0, ea1, g, G0, G1, G2, bg, ln_scale, ln_bias):
    def body(na_ref, ea0_ref, ea1_ref, g_ref, g0, g1, g2, bg_r, sc_ref,
             bi_ref, go_ref):
        ng = jnp.dot(na_ref[...], g0[...], preferred_element_type=jnp.float32)
        ng += jnp.dot(ea0_ref[...] + ea1_ref[...], g1[...],
                      preferred_element_type=jnp.float32)
        ng += jnp.dot(g_ref[...], g2[...], preferred_element_type=jnp.float32)
        ng = jnp.maximum(ng + bg_r[...], 0.0)
        x = ng + g_ref[...]
        mu = jnp.mean(x, axis=-1, keepdims=True)
        xc = x - mu
        var = jnp.mean(xc * xc, axis=-1, keepdims=True)
        go_ref[...] = xc * lax.rsqrt(var + LN_EPS) * sc_ref[...] + bi_ref[...]

    return pl.pallas_call(
        body, out_shape=jax.ShapeDtypeStruct((1, D), jnp.float32))(
            node_attr, ea0, ea1, g, G0, G1, G2, bg.reshape(1, D),
            ln_scale.reshape(1, D), ln_bias.reshape(1, D))


def _concat_rows_tc(a, b, W, bias, brows=2000):
    M = a.shape[0]
    K = a.shape[1]
    Dout = W.shape[1]
    nb = M // brows

    def body(a_ref, b_ref, w_ref, bias_ref, o_ref):
        i = pl.program_id(0)
        x = jnp.where(i < nb, a_ref[...], b_ref[...])
        o_ref[...] = jnp.dot(x, w_ref[...],
                             preferred_element_type=jnp.float32) + bias_ref[...]

    def amap(i):
        return (jnp.minimum(i, nb - 1), 0)

    def bmap(i):
        return (jnp.maximum(i - nb, 0), 0)

    return pl.pallas_call(
        body,
        grid=(2 * nb,),
        in_specs=[
            pl.BlockSpec((brows, K), amap),
            pl.BlockSpec((brows, K), bmap),
            pl.BlockSpec((K, Dout), lambda i: (0, 0)),
            pl.BlockSpec((1, Dout), lambda i: (0, 0)),
        ],
        out_specs=pl.BlockSpec((brows, Dout), lambda i: (i, 0)),
        out_shape=jax.ShapeDtypeStruct((2 * M, Dout), jnp.float32),
    )(a, b, W, bias.reshape(1, Dout))


_MESH = plsc.VectorSubcoreMesh(core_axis_name="c", subcore_axis_name="s",
                               num_cores=NC, num_subcores=NS)

_G_CHUNK = 128
_G_NCH = EH // _G_CHUNK
_G_PW = _G_NCH // NW
_G_EXTRA = _G_NCH - _G_PW * NW
_G_RING = 3


def _sc_gather_one(table_hbm, idx_hbm, out_hbm, ibufs, bufs, isems, gsems,
                   osems, wid):
    R = _G_RING
    F3 = (_G_PW // R) * R
    rem = _G_PW - F3

    def fire_i(j, b):
        c = (wid + j * NW) * _G_CHUNK
        pltpu.async_copy(idx_hbm.at[pl.ds(c, _G_CHUNK)], ibufs[b], isems[b])

    def wait_i(b):
        pltpu.make_async_copy(idx_hbm.at[pl.ds(0, _G_CHUNK)], ibufs[b],
                              isems[b]).wait()

    def fire_g(b):
        pltpu.async_copy(table_hbm.at[ibufs[b]], bufs[b], gsems[b])

    def wait_g(b):
        pltpu.make_async_copy(out_hbm.at[pl.ds(0, _G_CHUNK)], bufs[b],
                              gsems[b]).wait()

    def fire_o(j, b):
        c = (wid + j * NW) * _G_CHUNK
        pltpu.async_copy(bufs[b], out_hbm.at[pl.ds(c, _G_CHUNK)], osems[b])

    def wait_o(b):
        pltpu.make_async_copy(out_hbm.at[pl.ds(0, _G_CHUNK)], bufs[b],
                              osems[b]).wait()

    for b in range(R):
        fire_i(b, b)
    for b in range(R):
        wait_i(b)
        fire_g(b)

    def body(k, carry):
        for b in range(R):
            wait_g(b)
            fire_o(k * R + b, b)
        for b in range(R):
            wait_o(b)
            fire_i(k * R + b + R, b)
        for b in range(R):
            wait_i(b)
            fire_g(b)
        return carry

    lax.fori_loop(0, F3 // R - 1, body, 0)
    for b in range(R):
        wait_g(b)
        fire_o(F3 - R + b, b)
    for b in range(R):
        wait_o(b)
    for j in range(rem):
        fire_i(F3 + j, 0)
        wait_i(0)
        fire_g(0)
        wait_g(0)
        fire_o(F3 + j, 0)
        wait_o(0)

    @pl.when(wid < _G_EXTRA)
    def _():
        fire_i(_G_PW, 0)
        wait_i(0)
        fire_g(0)
        wait_g(0)
        fire_o(_G_PW, 0)
        wait_o(0)


_G_SCRATCH = [
    pltpu.VMEM((_G_CHUNK,), jnp.int32),
    pltpu.VMEM((_G_CHUNK,), jnp.int32),
    pltpu.VMEM((_G_CHUNK,), jnp.int32),
    pltpu.VMEM((_G_CHUNK, D), jnp.float32),
    pltpu.VMEM((_G_CHUNK, D), jnp.float32),
    pltpu.VMEM((_G_CHUNK, D), jnp.float32),
] + [pltpu.SemaphoreType.DMA] * 9


def _sc_gather2(table, senders, receivers):
    @functools.partial(
        pl.kernel,
        out_type=(jax.ShapeDtypeStruct((EH, D), jnp.float32),
                  jax.ShapeDtypeStruct((EH, D), jnp.float32)),
        mesh=_MESH,
        scratch_types=_G_SCRATCH,
    )
    def k(table_hbm, s_hbm, r_hbm, os_hbm, or_hbm, i0, i1, i2, b0, b1, b2,
          s0, s1, s2, g0, g1, g2, o0, o1, o2):
        wid = lax.axis_index("s") * NC + lax.axis_index("c")
        _sc_gather_one(table_hbm, s_hbm, os_hbm, (i0, i1, i2), (b0, b1, b2),
                       (s0, s1, s2), (g0, g1, g2), (o0, o1, o2), wid)
        _sc_gather_one(table_hbm, r_hbm, or_hbm, (i0, i1, i2), (b0, b1, b2),
                       (s0, s1, s2), (g0, g1, g2), (o0, o1, o2), wid)

    return k(table, senders, receivers)


def _sc_gather1(table, idx):
    @functools.partial(
        pl.kernel,
        out_type=jax.ShapeDtypeStruct((EH, D), jnp.float32),
        mesh=_MESH,
        scratch_types=_G_SCRATCH,
    )
    def k(table_hbm, i_hbm, out_hbm, i0, i1, i2, b0, b1, b2, s0, s1, s2, g0,
          g1, g2, o0, o1, o2):
        wid = lax.axis_index("s") * NC + lax.axis_index("c")
        _sc_gather_one(table_hbm, i_hbm, out_hbm, (i0, i1, i2), (b0, b1, b2),
                       (s0, s1, s2), (g0, g1, g2), (o0, o1, o2), wid)

    return k(table, idx)


_S_CHUNK = 40
N_PAD = 10240
_Z_ROWS = N_PAD // NS
_S_RING = 5
_S_SEC = 5
_S_NCH = (EH // NS) // _S_CHUNK
_S_SECN = _S_NCH // _S_SEC


def _sc_scatter_loop(val_hbm, base, slab_fn, idx_slab, vbufs, vsems, ssems,
                     acc):
    R = _S_RING

    def fire_v(c, b):
        pltpu.async_copy(val_hbm.at[pl.ds(base + c * _S_CHUNK, _S_CHUNK)],
                         vbufs[b], vsems[b])

    def wait_v(b):
        pltpu.make_async_copy(val_hbm.at[pl.ds(base, _S_CHUNK)], vbufs[b],
                              vsems[b]).wait()

    def fire_s(j, b):
        pltpu.async_copy(vbufs[b], acc.at[idx_slab.at[j]], ssems[b], add=True)

    def wait_s(b):
        pltpu.make_async_copy(val_hbm.at[pl.ds(base, _S_CHUNK)], vbufs[b],
                              ssems[b]).wait()

    def section(s, carry):
        pltpu.sync_copy(slab_fn(s), idx_slab)
        c0 = s * _S_SECN
        for b in range(R):
            fire_v(c0 + b, b)

        def body(k, carry2):
            for b in range(R):
                wait_v(b)
                fire_s(k * R + b, b)
            for b in range(R):
                wait_s(b)
                fire_v(c0 + k * R + b + R, b)
            return carry2

        lax.fori_loop(0, _S_SECN // R - 1, body, 0)
        for b in range(R):
            wait_v(b)
            fire_s(_S_SECN - R + b, b)
        for b in range(R):
            wait_s(b)
        return carry

    lax.fori_loop(0, _S_SEC, section, 0)


_S_SCRATCH = [
    pltpu.VMEM((_S_SECN, _S_CHUNK), jnp.int32),
    pltpu.VMEM((_S_CHUNK, D), jnp.float32),
    pltpu.VMEM((_S_CHUNK, D), jnp.float32),
    pltpu.VMEM((_S_CHUNK, D), jnp.float32),
    pltpu.VMEM((_S_CHUNK, D), jnp.float32),
    pltpu.VMEM((_S_CHUNK, D), jnp.float32),
] + [pltpu.SemaphoreType.DMA] * 10 + [
    pltpu.VMEM_SHARED((N_PAD, D), jnp.float32),
]


def _sc_segsum_ex(ex0, ex1, ridx6, zeros):
    @functools.partial(
        pl.kernel,
        out_type=jax.ShapeDtypeStruct((NC * N_PAD, D), jnp.float32),
        mesh=_MESH,
        scratch_types=_S_SCRATCH,
    )
    def k(e0_hbm, e1_hbm, idx6_hbm, z_hbm, out_hbm, idx_slab, v0, v1, v2, v3,
          v4, s0, s1, s2, s3, s4, t0, t1, t2, t3, t4, acc):
        cid = lax.axis_index("c")
        sid = lax.axis_index("s")
        pltpu.sync_copy(z_hbm, acc.at[pl.ds(sid * _Z_ROWS, _Z_ROWS)])
        plsc.subcore_barrier()
        vbufs = (v0, v1, v2, v3, v4)
        vsems = (s0, s1, s2, s3, s4)
        ssems = (t0, t1, t2, t3, t4)
        base = sid * (EH // NS)

        @pl.when(cid == 0)
        def _():
            _sc_scatter_loop(e0_hbm, base, lambda s: idx6_hbm.at[0, sid, s],
                             idx_slab, vbufs, vsems, ssems, acc)

        @pl.when(cid == 1)
        def _():
            _sc_scatter_loop(e1_hbm, base, lambda s: idx6_hbm.at[1, sid, s],
                             idx_slab, vbufs, vsems, ssems, acc)

        plsc.subcore_barrier()
        pltpu.sync_copy(acc.at[pl.ds(sid * _Z_ROWS, _Z_ROWS)],
                        out_hbm.at[pl.ds(cid * N_PAD + sid * _Z_ROWS,
                                         _Z_ROWS)])

    return k(ex0, ex1, ridx6, zeros)


def _sc_segsum_sr(att, sidx5, ridx5, zeros):
    @functools.partial(
        pl.kernel,
        out_type=jax.ShapeDtypeStruct((NC * N_PAD, D), jnp.float32),
        mesh=_MESH,
        scratch_types=_S_SCRATCH,
    )
    def k(a_hbm, sidx_hbm, ridx_hbm, z_hbm, out_hbm, idx_slab, v0, v1, v2,
          v3, v4, s0, s1, s2, s3, s4, t0, t1, t2, t3, t4, acc):
        cid = lax.axis_index("c")
        sid = lax.axis_index("s")
        pltpu.sync_copy(z_hbm, acc.at[pl.ds(sid * _Z_ROWS, _Z_ROWS)])
        plsc.subcore_barrier()
        vbufs = (v0, v1, v2, v3, v4)
        vsems = (s0, s1, s2, s3, s4)
        ssems = (t0, t1, t2, t3, t4)
        base = sid * (EH // NS)

        @pl.when(cid == 0)
        def _():
            _sc_scatter_loop(a_hbm, base, lambda s: sidx_hbm.at[sid, s],
                             idx_slab, vbufs, vsems, ssems, acc)

        @pl.when(cid == 1)
        def _():
            _sc_scatter_loop(a_hbm, base, lambda s: ridx_hbm.at[sid, s],
                             idx_slab, vbufs, vsems, ssems, acc)

        plsc.subcore_barrier()
        pltpu.sync_copy(acc.at[pl.ds(sid * _Z_ROWS, _Z_ROWS)],
                        out_hbm.at[pl.ds(cid * N_PAD + sid * _Z_ROWS,
                                         _Z_ROWS)])

    return k(att, sidx5, ridx5, zeros)


def kernel(nodes, edges, globals_, senders, receivers, params):
    zeros = jnp.zeros((_Z_ROWS, D), jnp.float32)
    s_h = (senders[:EH], senders[EH:])
    r_h = (receivers[:EH], receivers[EH:])
    ridx6 = receivers.reshape(2, NS, _S_SEC, _S_SECN, _S_CHUNK)
    sidx6 = senders.reshape(2, NS, _S_SEC, _S_SECN, _S_CHUNK)

    nlat = _dense_tc(nodes, params["embed_node"]["W"],
                     params["embed_node"]["b"], brows=2000)
    elat = [
        _dense_tc(edges, params["embed_edge"]["W"], params["embed_edge"]["b"],
                  brows=2000, in_off=h * (EH // 2000), rows=EH)
        for h in range(2)
    ]
    g = _dense_tc(globals_, params["embed_global"]["W"],
                  params["embed_global"]["b"], brows=1)

    for s in range(2):
        sp = params["step%d" % s]
        We, be = sp["edge_mlp"][0]["W"], sp["edge_mlp"][0]["b"]
        Wa, ba = sp["attn_mlp"][0]["W"], sp["attn_mlp"][0]["b"]
        Wn, bn = sp["node_mlp"][0]["W"], sp["node_mlp"][0]["b"]
        Wg, bg = sp["global_mlp"][0]["W"], sp["global_mlp"][0]["b"]

        gew, gaw, gnw = _prep_tc(g, We[384:512], be, Wa[128:256], ba,
                                 Wn[384:512], bn)
        sr = [_sc_gather2(nlat, s_h[h], r_h[h]) for h in range(2)]
        ex0, p0 = _edge_fused_tc(elat[0], sr[0][0], sr[0][1], We[0:128],
                                 We[128:256], We[256:384], Wa[0:128], gew,
                                 gaw)
        ex1, p1 = _edge_fused_tc(elat[1], sr[1][0], sr[1][1], We[0:128],
                                 We[128:256], We[256:384], Wa[0:128], gew,
                                 gaw)
        dq = _sc_segsum_ex(ex0, ex1, ridx6, zeros).reshape(NC, N_PAD, D)
        rinv = _nd_combine_tc(dq)
        rr0 = _sc_gather1(rinv, r_h[0])
        rr1 = _sc_gather1(rinv, r_h[1])
        att0, el0, ea0 = _att_fused_tc(p0, rr0, elat[0],
                                       sp["ln_edges"]["scale"],
                                       sp["ln_edges"]["bias"])
        spa = _sc_segsum_sr(att0, sidx6[0], ridx6[0],
                            zeros).reshape(NC, N_PAD, D)
        att1, el1, ea1 = _att_fused_tc(p1, rr1, elat[1],
                                       sp["ln_edges"]["scale"],
                                       sp["ln_edges"]["bias"])
        spb = _sc_segsum_sr(att1, sidx6[1], ridx6[1],
                            zeros).reshape(NC, N_PAD, D)
        elat = [el0, el1]
        aee = ((att0, el0, ea0), (att1, el1, ea1))
        nlat, node_attr = _node_fused_tc(
            nlat, spa, spb, Wn[0:128], Wn[128:256], Wn[256:384], gnw,
            sp["ln_nodes"]["scale"], sp["ln_nodes"]["bias"])
        g = _gup_tc(node_attr, aee[0][2], aee[1][2], g, Wg[0:128],
                    Wg[128:256], Wg[256:384], bg, sp["ln_globals"]["scale"],
                    sp["ln_globals"]["bias"])

    nodes_o = _dense_tc(nlat, params["decode_node"]["W"],
                        params["decode_node"]["b"], brows=2000)
    edges_o = _concat_rows_tc(elat[0], elat[1], params["decode_edge"]["W"],
                              params["decode_edge"]["b"], brows=2000)
    g_o = _dense_tc(g, params["decode_global"]["W"],
                    params["decode_global"]["b"], brows=1)
    return nodes_o, edges_o, g_o


# speedup vs baseline: 1.0849x; 1.0849x over previous
import functools

import jax
import jax.numpy as jnp
from jax import lax
from jax.experimental import pallas as pl
from jax.experimental.pallas import tpu as pltpu
from jax.experimental.pallas import tpu_sc as plsc

N_NODES = 10000
N_EDGES = 160000
EH = N_EDGES // 2
D = 128
LN_EPS = 1e-6

NC = 2
NS = 16
NW = NC * NS


def _dense_tc(x, W, b, brows, in_off=0, rows=None):
    M, K = x.shape
    rows = rows if rows is not None else M
    Dout = W.shape[1]
    b2 = b.reshape(1, Dout)

    def body(x_ref, w_ref, b_ref, o_ref):
        y = jnp.dot(x_ref[...], w_ref[...],
                    preferred_element_type=jnp.float32) + b_ref[...]
        o_ref[...] = y

    return pl.pallas_call(
        body,
        grid=(rows // brows,),
        in_specs=[
            pl.BlockSpec((brows, K), lambda i: (i + in_off, 0)),
            pl.BlockSpec((K, Dout), lambda i: (0, 0)),
            pl.BlockSpec((1, Dout), lambda i: (0, 0)),
        ],
        out_specs=pl.BlockSpec((brows, Dout), lambda i: (i, 0)),
        out_shape=jax.ShapeDtypeStruct((rows, Dout), jnp.float32),
    )(x, W, b2)


def _prep_tc(g, Weg, be, Wag, ba, Wng, bn):
    def body(g_ref, weg, be_r, wag, ba_r, wng, bn_r, gew, gaw, gnw):
        gv = g_ref[...]
        gew[...] = jnp.dot(gv, weg[...], preferred_element_type=jnp.float32) + be_r[...]
        gaw[...] = jnp.dot(gv, wag[...], preferred_element_type=jnp.float32) + ba_r[...]
        gnw[...] = jnp.dot(gv, wng[...], preferred_element_type=jnp.float32) + bn_r[...]

    out = jax.ShapeDtypeStruct((1, D), jnp.float32)
    return pl.pallas_call(body, out_shape=(out, out, out))(
        g, Weg, be.reshape(1, D), Wag, ba.reshape(1, D), Wng, bn.reshape(1, D))


def _edge_fused_tc(elat, sent, recv, W0, W1, W2, A0, gew, gaw, brows=2000):
    M = elat.shape[0]

    def body(e_ref, s_ref, r_ref, w0, w1, w2, a0, gew_r, gaw_r, ex_ref, p_ref):
        ne = jnp.dot(e_ref[...].astype(jnp.bfloat16), w0[...],
                     preferred_element_type=jnp.float32)
        ne += jnp.dot(s_ref[...].astype(jnp.bfloat16), w1[...],
                      preferred_element_type=jnp.float32)
        ne += jnp.dot(r_ref[...].astype(jnp.bfloat16), w2[...],
                      preferred_element_type=jnp.float32)
        ne = jnp.maximum(ne + gew_r[...], 0.0)
        lg = jnp.dot(ne.astype(jnp.bfloat16), a0[...],
                     preferred_element_type=jnp.float32)
        lg = jnp.maximum(lg + gaw_r[...], 0.0)
        ex = jnp.exp(lg)
        ex_ref[...] = ex
        p_ref[...] = ex * ne

    blk = pl.BlockSpec((brows, D), lambda i: (i, 0))
    wblk = pl.BlockSpec((D, D), lambda i: (0, 0))
    vblk = pl.BlockSpec((1, D), lambda i: (0, 0))
    out = jax.ShapeDtypeStruct((M, D), jnp.float32)
    bf = jnp.bfloat16
    return pl.pallas_call(
        body,
        grid=(M // brows,),
        in_specs=[blk, blk, blk, wblk, wblk, wblk, wblk, vblk, vblk],
        out_specs=(blk, blk),
        out_shape=(out, out),
    )(elat, sent, recv, W0.astype(bf), W1.astype(bf), W2.astype(bf),
      A0.astype(bf), gew, gaw)


def _nd_combine_tc(dq, brows=2048):
    N = dq.shape[1]

    def body(a_ref, ri_ref):
        d = a_ref[0] + a_ref[1]
        ri_ref[...] = 1.0 / jnp.where(d == 0.0, 1.0, d)

    blk3 = pl.BlockSpec((2, brows, D), lambda i: (0, i, 0))
    blk = pl.BlockSpec((brows, D), lambda i: (i, 0))
    return pl.pallas_call(
        body,
        grid=(N // brows,),
        in_specs=[blk3],
        out_specs=blk,
        out_shape=jax.ShapeDtypeStruct((N, D), jnp.float32),
    )(dq)


def _att_fused_tc(p, rinvr, elat, ln_scale, ln_bias, brows=2000):
    M = p.shape[0]

    def body(p_ref, dr_ref, el_ref, sc_ref, bi_ref, att_ref, eo_ref, ea_ref):
        att = p_ref[...] * dr_ref[...]
        att_ref[...] = att
        x = att + el_ref[...]
        mu = jnp.mean(x, axis=-1, keepdims=True)
        xc = x - mu
        var = jnp.mean(xc * xc, axis=-1, keepdims=True)
        eo_ref[...] = xc * lax.rsqrt(var + LN_EPS) * sc_ref[...] + bi_ref[...]

        @pl.when(pl.program_id(0) == 0)
        def _():
            ea_ref[...] = jnp.zeros_like(ea_ref)

        ea_ref[...] += jnp.sum(att, axis=0, keepdims=True)

    blk = pl.BlockSpec((brows, D), lambda i: (i, 0))
    vblk = pl.BlockSpec((1, D), lambda i: (0, 0))
    out = jax.ShapeDtypeStruct((M, D), jnp.float32)
    outv = jax.ShapeDtypeStruct((1, D), jnp.float32)
    return pl.pallas_call(
        body,
        grid=(M // brows,),
        in_specs=[blk, blk, blk, vblk, vblk],
        out_specs=(blk, blk, vblk),
        out_shape=(out, out, outv),
    )(p, rinvr, elat, ln_scale.reshape(1, D), ln_bias.reshape(1, D))


def _node_fused_tc(nlat, spa, spb, Wn0, Wn1, Wn2, gnw, ln_scale,
                   ln_bias, brows=2000):
    N = nlat.shape[0]

    def body(n_ref, spa_ref, spb_ref, w0, w1, w2, gnw_r, sc_ref, bi_ref,
             no_ref, na_ref):
        sa = spa_ref[0] + spb_ref[0]
        rv = spa_ref[1] + spb_ref[1]
        nn = jnp.dot(n_ref[...], w0[...], preferred_element_type=jnp.float32)
        nn += jnp.dot(sa, w1[...], preferred_element_type=jnp.float32)
        nn += jnp.dot(rv, w2[...], preferred_element_type=jnp.float32)
        nn = jnp.maximum(nn + gnw_r[...], 0.0)

        @pl.when(pl.program_id(0) == 0)
        def _():
            na_ref[...] = jnp.zeros_like(na_ref)

        na_ref[...] += jnp.sum(nn, axis=0, keepdims=True)
        x = nn + n_ref[...]
        mu = jnp.mean(x, axis=-1, keepdims=True)
        xc = x - mu
        var = jnp.mean(xc * xc, axis=-1, keepdims=True)
        no_ref[...] = xc * lax.rsqrt(var + LN_EPS) * sc_ref[...] + bi_ref[...]

    blk = pl.BlockSpec((brows, D), lambda i: (i, 0))
    blk3 = pl.BlockSpec((2, brows, D), lambda i: (0, i, 0))
    wblk = pl.BlockSpec((D, D), lambda i: (0, 0))
    vblk = pl.BlockSpec((1, D), lambda i: (0, 0))
    return pl.pallas_call(
        body,
        grid=(N // brows,),
        in_specs=[blk, blk3, blk3, wblk, wblk, wblk, vblk, vblk, vblk],
        out_specs=(blk, vblk),
        out_shape=(jax.ShapeDtypeStruct((N, D), jnp.float32),
                   jax.ShapeDtypeStruct((1, D), jnp.float32)),
    )(nlat, spa, spb, Wn0, Wn1, Wn2, gnw,
      ln_scale.reshape(1, D), ln_bias.reshape(1, D))


def _gup_tc(node_attr, ea0, ea1, g, G0, G1, G2, bg, ln_scale, ln_bias):
    def body(na_ref, ea0_ref, ea1_ref, g_ref, g0, g1, g2, bg_r, sc_ref,
             bi_ref, go_ref):
        ng = jnp.dot(na_ref[...], g0[...], preferred_element_type=jnp.float32)
        ng += jnp.dot(ea0_ref[...] + ea1_ref[...], g1[...],
                      preferred_element_type=jnp.float32)
        ng += jnp.dot(g_ref[...], g2[...], preferred_element_type=jnp.float32)
        ng = jnp.maximum(ng + bg_r[...], 0.0)
        x = ng + g_ref[...]
        mu = jnp.mean(x, axis=-1, keepdims=True)
        xc = x - mu
        var = jnp.mean(xc * xc, axis=-1, keepdims=True)
        go_ref[...] = xc * lax.rsqrt(var + LN_EPS) * sc_ref[...] + bi_ref[...]

    return pl.pallas_call(
        body, out_shape=jax.ShapeDtypeStruct((1, D), jnp.float32))(
            node_attr, ea0, ea1, g, G0, G1, G2, bg.reshape(1, D),
            ln_scale.reshape(1, D), ln_bias.reshape(1, D))


def _concat_rows_tc(a, b, W, bias, brows=2000):
    M = a.shape[0]
    K = a.shape[1]
    Dout = W.shape[1]
    nb = M // brows

    def body(a_ref, b_ref, w_ref, bias_ref, o_ref):
        i = pl.program_id(0)
        x = jnp.where(i < nb, a_ref[...], b_ref[...])
        o_ref[...] = jnp.dot(x, w_ref[...],
                             preferred_element_type=jnp.float32) + bias_ref[...]

    def amap(i):
        return (jnp.minimum(i, nb - 1), 0)

    def bmap(i):
        return (jnp.maximum(i - nb, 0), 0)

    return pl.pallas_call(
        body,
        grid=(2 * nb,),
        in_specs=[
            pl.BlockSpec((brows, K), amap),
            pl.BlockSpec((brows, K), bmap),
            pl.BlockSpec((K, Dout), lambda i: (0, 0)),
            pl.BlockSpec((1, Dout), lambda i: (0, 0)),
        ],
        out_specs=pl.BlockSpec((brows, Dout), lambda i: (i, 0)),
        out_shape=jax.ShapeDtypeStruct((2 * M, Dout), jnp.float32),
    )(a, b, W, bias.reshape(1, Dout))


_MESH = plsc.VectorSubcoreMesh(core_axis_name="c", subcore_axis_name="s",
                               num_cores=NC, num_subcores=NS)

_G_CHUNK = 128
_G_NCH = EH // _G_CHUNK
_G_PW = _G_NCH // NW
_G_EXTRA = _G_NCH - _G_PW * NW
_G_RING = 3


def _sc_gather_one(table_hbm, idx_hbm, out_hbm, ibufs, bufs, isems, gsems,
                   osems, wid):
    R = _G_RING
    F3 = (_G_PW // R) * R
    rem = _G_PW - F3

    def fire_i(j, b):
        c = (wid + j * NW) * _G_CHUNK
        pltpu.async_copy(idx_hbm.at[pl.ds(c, _G_CHUNK)], ibufs[b], isems[b])

    def wait_i(b):
        pltpu.make_async_copy(idx_hbm.at[pl.ds(0, _G_CHUNK)], ibufs[b],
                              isems[b]).wait()

    def fire_g(b):
        pltpu.async_copy(table_hbm.at[ibufs[b]], bufs[b], gsems[b])

    def wait_g(b):
        pltpu.make_async_copy(out_hbm.at[pl.ds(0, _G_CHUNK)], bufs[b],
                              gsems[b]).wait()

    def fire_o(j, b):
        c = (wid + j * NW) * _G_CHUNK
        pltpu.async_copy(bufs[b], out_hbm.at[pl.ds(c, _G_CHUNK)], osems[b])

    def wait_o(b):
        pltpu.make_async_copy(out_hbm.at[pl.ds(0, _G_CHUNK)], bufs[b],
                              osems[b]).wait()

    for b in range(R):
        fire_i(b, b)
    for b in range(R):
        wait_i(b)
        fire_g(b)

    def body(k, carry):
        for b in range(R):
            wait_g(b)
            fire_o(k * R + b, b)
        for b in range(R):
            wait_o(b)
            fire_i(k * R + b + R, b)
        for b in range(R):
            wait_i(b)
            fire_g(b)
        return carry

    lax.fori_loop(0, F3 // R - 1, body, 0)
    for b in range(R):
        wait_g(b)
        fire_o(F3 - R + b, b)
    for b in range(R):
        wait_o(b)
    for j in range(rem):
        fire_i(F3 + j, 0)
        wait_i(0)
        fire_g(0)
        wait_g(0)
        fire_o(F3 + j, 0)
        wait_o(0)

    @pl.when(wid < _G_EXTRA)
    def _():
        fire_i(_G_PW, 0)
        wait_i(0)
        fire_g(0)
        wait_g(0)
        fire_o(_G_PW, 0)
        wait_o(0)


_G_SCRATCH = [
    pltpu.VMEM((_G_CHUNK,), jnp.int32),
    pltpu.VMEM((_G_CHUNK,), jnp.int32),
    pltpu.VMEM((_G_CHUNK,), jnp.int32),
    pltpu.VMEM((_G_CHUNK, D), jnp.float32),
    pltpu.VMEM((_G_CHUNK, D), jnp.float32),
    pltpu.VMEM((_G_CHUNK, D), jnp.float32),
] + [pltpu.SemaphoreType.DMA] * 9 + [
    pltpu.VMEM_SHARED((N_NODES, D), jnp.float32),
]

_T_ROWS = 632


def _stage_table(table_hbm, tbl, sid):
    @pl.when(sid < NS - 1)
    def _():
        pltpu.sync_copy(table_hbm.at[pl.ds(sid * _T_ROWS, _T_ROWS)],
                        tbl.at[pl.ds(sid * _T_ROWS, _T_ROWS)])

    @pl.when(sid == NS - 1)
    def _():
        last = N_NODES - (NS - 1) * _T_ROWS
        pltpu.sync_copy(table_hbm.at[pl.ds((NS - 1) * _T_ROWS, last)],
                        tbl.at[pl.ds((NS - 1) * _T_ROWS, last)])

    plsc.subcore_barrier()


def _sc_gather2(table, senders, receivers):
    @functools.partial(
        pl.kernel,
        out_type=(jax.ShapeDtypeStruct((EH, D), jnp.float32),
                  jax.ShapeDtypeStruct((EH, D), jnp.float32)),
        mesh=_MESH,
        scratch_types=_G_SCRATCH,
    )
    def k(table_hbm, s_hbm, r_hbm, os_hbm, or_hbm, i0, i1, i2, b0, b1, b2,
          s0, s1, s2, g0, g1, g2, o0, o1, o2, tbl):
        cid = lax.axis_index("c")
        sid = lax.axis_index("s")
        wid = sid * NC + cid
        _stage_table(table_hbm, tbl, sid)
        _sc_gather_one(tbl, s_hbm, os_hbm, (i0, i1, i2), (b0, b1, b2),
                       (s0, s1, s2), (g0, g1, g2), (o0, o1, o2), wid)
        _sc_gather_one(tbl, r_hbm, or_hbm, (i0, i1, i2), (b0, b1, b2),
                       (s0, s1, s2), (g0, g1, g2), (o0, o1, o2), wid)

    return k(table, senders, receivers)


def _sc_gather1(table, idx):
    @functools.partial(
        pl.kernel,
        out_type=jax.ShapeDtypeStruct((EH, D), jnp.float32),
        mesh=_MESH,
        scratch_types=_G_SCRATCH,
    )
    def k(table_hbm, i_hbm, out_hbm, i0, i1, i2, b0, b1, b2, s0, s1, s2, g0,
          g1, g2, o0, o1, o2, tbl):
        cid = lax.axis_index("c")
        sid = lax.axis_index("s")
        wid = sid * NC + cid
        _stage_table(table_hbm, tbl, sid)
        _sc_gather_one(tbl, i_hbm, out_hbm, (i0, i1, i2), (b0, b1, b2),
                       (s0, s1, s2), (g0, g1, g2), (o0, o1, o2), wid)

    return k(table, idx)


_S_CHUNK = 40
N_PAD = 10240
_Z_ROWS = N_PAD // NS
_S_RING = 5
_S_SEC = 5
_S_NCH = (EH // NS) // _S_CHUNK
_S_SECN = _S_NCH // _S_SEC


def _sc_scatter_loop(val_hbm, base, slab_fn, idx_slab, vbufs, vsems, ssems,
                     acc):
    R = _S_RING

    def fire_v(c, b):
        pltpu.async_copy(val_hbm.at[pl.ds(base + c * _S_CHUNK, _S_CHUNK)],
                         vbufs[b], vsems[b])

    def wait_v(b):
        pltpu.make_async_copy(val_hbm.at[pl.ds(base, _S_CHUNK)], vbufs[b],
                              vsems[b]).wait()

    def fire_s(j, b):
        pltpu.async_copy(vbufs[b], acc.at[idx_slab.at[j]], ssems[b], add=True)

    def wait_s(b):
        pltpu.make_async_copy(val_hbm.at[pl.ds(base, _S_CHUNK)], vbufs[b],
                              ssems[b]).wait()

    def section(s, carry):
        pltpu.sync_copy(slab_fn(s), idx_slab)
        c0 = s * _S_SECN
        for b in range(R):
            fire_v(c0 + b, b)

        def body(k, carry2):
            for b in range(R):
                wait_v(b)
                fire_s(k * R + b, b)
            for b in range(R):
                wait_s(b)
                fire_v(c0 + k * R + b + R, b)
            return carry2

        lax.fori_loop(0, _S_SECN // R - 1, body, 0)
        for b in range(R):
            wait_v(b)
            fire_s(_S_SECN - R + b, b)
        for b in range(R):
            wait_s(b)
        return carry

    lax.fori_loop(0, _S_SEC, section, 0)


_S_SCRATCH = [
    pltpu.VMEM((_S_SECN, _S_CHUNK), jnp.int32),
    pltpu.VMEM((_S_CHUNK, D), jnp.float32),
    pltpu.VMEM((_S_CHUNK, D), jnp.float32),
    pltpu.VMEM((_S_CHUNK, D), jnp.float32),
    pltpu.VMEM((_S_CHUNK, D), jnp.float32),
    pltpu.VMEM((_S_CHUNK, D), jnp.float32),
] + [pltpu.SemaphoreType.DMA] * 10 + [
    pltpu.VMEM_SHARED((N_PAD, D), jnp.float32),
]


def _sc_segsum_ex(ex0, ex1, ridx6, zeros):
    @functools.partial(
        pl.kernel,
        out_type=jax.ShapeDtypeStruct((NC * N_PAD, D), jnp.float32),
        mesh=_MESH,
        scratch_types=_S_SCRATCH,
    )
    def k(e0_hbm, e1_hbm, idx6_hbm, z_hbm, out_hbm, idx_slab, v0, v1, v2, v3,
          v4, s0, s1, s2, s3, s4, t0, t1, t2, t3, t4, acc):
        cid = lax.axis_index("c")
        sid = lax.axis_index("s")
        pltpu.sync_copy(z_hbm, acc.at[pl.ds(sid * _Z_ROWS, _Z_ROWS)])
        plsc.subcore_barrier()
        vbufs = (v0, v1, v2, v3, v4)
        vsems = (s0, s1, s2, s3, s4)
        ssems = (t0, t1, t2, t3, t4)
        base = sid * (EH // NS)

        @pl.when(cid == 0)
        def _():
            _sc_scatter_loop(e0_hbm, base, lambda s: idx6_hbm.at[0, sid, s],
                             idx_slab, vbufs, vsems, ssems, acc)

        @pl.when(cid == 1)
        def _():
            _sc_scatter_loop(e1_hbm, base, lambda s: idx6_hbm.at[1, sid, s],
                             idx_slab, vbufs, vsems, ssems, acc)

        plsc.subcore_barrier()
        pltpu.sync_copy(acc.at[pl.ds(sid * _Z_ROWS, _Z_ROWS)],
                        out_hbm.at[pl.ds(cid * N_PAD + sid * _Z_ROWS,
                                         _Z_ROWS)])

    return k(ex0, ex1, ridx6, zeros)


def _sc_segsum_sr(att, sidx5, ridx5, zeros):
    @functools.partial(
        pl.kernel,
        out_type=jax.ShapeDtypeStruct((NC * N_PAD, D), jnp.float32),
        mesh=_MESH,
        scratch_types=_S_SCRATCH,
    )
    def k(a_hbm, sidx_hbm, ridx_hbm, z_hbm, out_hbm, idx_slab, v0, v1, v2,
          v3, v4, s0, s1, s2, s3, s4, t0, t1, t2, t3, t4, acc):
        cid = lax.axis_index("c")
        sid = lax.axis_index("s")
        pltpu.sync_copy(z_hbm, acc.at[pl.ds(sid * _Z_ROWS, _Z_ROWS)])
        plsc.subcore_barrier()
        vbufs = (v0, v1, v2, v3, v4)
        vsems = (s0, s1, s2, s3, s4)
        ssems = (t0, t1, t2, t3, t4)
        base = sid * (EH // NS)

        @pl.when(cid == 0)
        def _():
            _sc_scatter_loop(a_hbm, base, lambda s: sidx_hbm.at[sid, s],
                             idx_slab, vbufs, vsems, ssems, acc)

        @pl.when(cid == 1)
        def _():
            _sc_scatter_loop(a_hbm, base, lambda s: ridx_hbm.at[sid, s],
                             idx_slab, vbufs, vsems, ssems, acc)

        plsc.subcore_barrier()
        pltpu.sync_copy(acc.at[pl.ds(sid * _Z_ROWS, _Z_ROWS)],
                        out_hbm.at[pl.ds(cid * N_PAD + sid * _Z_ROWS,
                                         _Z_ROWS)])

    return k(att, sidx5, ridx5, zeros)


def kernel(nodes, edges, globals_, senders, receivers, params):
    zeros = jnp.zeros((_Z_ROWS, D), jnp.float32)
    s_h = (senders[:EH], senders[EH:])
    r_h = (receivers[:EH], receivers[EH:])
    ridx6 = receivers.reshape(2, NS, _S_SEC, _S_SECN, _S_CHUNK)
    sidx6 = senders.reshape(2, NS, _S_SEC, _S_SECN, _S_CHUNK)

    nlat = _dense_tc(nodes, params["embed_node"]["W"],
                     params["embed_node"]["b"], brows=2000)
    elat = [
        _dense_tc(edges, params["embed_edge"]["W"], params["embed_edge"]["b"],
                  brows=2000, in_off=h * (EH // 2000), rows=EH)
        for h in range(2)
    ]
    g = _dense_tc(globals_, params["embed_global"]["W"],
                  params["embed_global"]["b"], brows=1)

    for s in range(2):
        sp = params["step%d" % s]
        We, be = sp["edge_mlp"][0]["W"], sp["edge_mlp"][0]["b"]
        Wa, ba = sp["attn_mlp"][0]["W"], sp["attn_mlp"][0]["b"]
        Wn, bn = sp["node_mlp"][0]["W"], sp["node_mlp"][0]["b"]
        Wg, bg = sp["global_mlp"][0]["W"], sp["global_mlp"][0]["b"]

        gew, gaw, gnw = _prep_tc(g, We[384:512], be, Wa[128:256], ba,
                                 Wn[384:512], bn)
        sr = [_sc_gather2(nlat, s_h[h], r_h[h]) for h in range(2)]
        ex0, p0 = _edge_fused_tc(elat[0], sr[0][0], sr[0][1], We[0:128],
                                 We[128:256], We[256:384], Wa[0:128], gew,
                                 gaw)
        ex1, p1 = _edge_fused_tc(elat[1], sr[1][0], sr[1][1], We[0:128],
                                 We[128:256], We[256:384], Wa[0:128], gew,
                                 gaw)
        dq = _sc_segsum_ex(ex0, ex1, ridx6, zeros).reshape(NC, N_PAD, D)
        rinv = _nd_combine_tc(dq)
        rr0 = _sc_gather1(rinv, r_h[0])
        rr1 = _sc_gather1(rinv, r_h[1])
        att0, el0, ea0 = _att_fused_tc(p0, rr0, elat[0],
                                       sp["ln_edges"]["scale"],
                                       sp["ln_edges"]["bias"])
        spa = _sc_segsum_sr(att0, sidx6[0], ridx6[0],
                            zeros).reshape(NC, N_PAD, D)
        att1, el1, ea1 = _att_fused_tc(p1, rr1, elat[1],
                                       sp["ln_edges"]["scale"],
                                       sp["ln_edges"]["bias"])
        spb = _sc_segsum_sr(att1, sidx6[1], ridx6[1],
                            zeros).reshape(NC, N_PAD, D)
        elat = [el0, el1]
        aee = ((att0, el0, ea0), (att1, el1, ea1))
        nlat, node_attr = _node_fused_tc(
            nlat, spa, spb, Wn[0:128], Wn[128:256], Wn[256:384], gnw,
            sp["ln_nodes"]["scale"], sp["ln_nodes"]["bias"])
        g = _gup_tc(node_attr, aee[0][2], aee[1][2], g, Wg[0:128],
                    Wg[128:256], Wg[256:384], bg, sp["ln_globals"]["scale"],
                    sp["ln_globals"]["bias"])

    nodes_o = _dense_tc(nlat, params["decode_node"]["W"],
                        params["decode_node"]["b"], brows=2000)
    edges_o = _concat_rows_tc(elat[0], elat[1], params["decode_edge"]["W"],
                              params["decode_edge"]["b"], brows=2000)
    g_o = _dense_tc(g, params["decode_global"]["W"],
                    params["decode_global"]["b"], brows=1)
    return nodes_o, edges_o, g_o


# speedup vs baseline: 1.1368x; 1.0478x over previous
import functools

import jax
import jax.numpy as jnp
from jax import lax
from jax.experimental import pallas as pl
from jax.experimental.pallas import tpu as pltpu
from jax.experimental.pallas import tpu_sc as plsc

N_NODES = 10000
N_EDGES = 160000
EH = N_EDGES // 2
D = 128
LN_EPS = 1e-6

NC = 2
NS = 16
NW = NC * NS


def _dense_tc(x, W, b, brows, in_off=0, rows=None):
    M, K = x.shape
    rows = rows if rows is not None else M
    Dout = W.shape[1]
    b2 = b.reshape(1, Dout)

    def body(x_ref, w_ref, b_ref, o_ref):
        y = jnp.dot(x_ref[...], w_ref[...],
                    preferred_element_type=jnp.float32) + b_ref[...]
        o_ref[...] = y

    return pl.pallas_call(
        body,
        grid=(rows // brows,),
        in_specs=[
            pl.BlockSpec((brows, K), lambda i: (i + in_off, 0)),
            pl.BlockSpec((K, Dout), lambda i: (0, 0)),
            pl.BlockSpec((1, Dout), lambda i: (0, 0)),
        ],
        out_specs=pl.BlockSpec((brows, Dout), lambda i: (i, 0)),
        out_shape=jax.ShapeDtypeStruct((rows, Dout), jnp.float32),
    )(x, W, b2)


def _prep_tc(g, Weg, be, Wag, ba, Wng, bn):
    def body(g_ref, weg, be_r, wag, ba_r, wng, bn_r, gew, gaw, gnw):
        gv = g_ref[...]
        gew[...] = jnp.dot(gv, weg[...], preferred_element_type=jnp.float32) + be_r[...]
        gaw[...] = jnp.dot(gv, wag[...], preferred_element_type=jnp.float32) + ba_r[...]
        gnw[...] = jnp.dot(gv, wng[...], preferred_element_type=jnp.float32) + bn_r[...]

    out = jax.ShapeDtypeStruct((1, D), jnp.float32)
    return pl.pallas_call(body, out_shape=(out, out, out))(
        g, Weg, be.reshape(1, D), Wag, ba.reshape(1, D), Wng, bn.reshape(1, D))


def _edge_fused_tc(elat, sent, recv, W0, W1, W2, A0, gew, gaw, brows=4000):
    M = elat.shape[0]

    def body(e_ref, s_ref, r_ref, w0, w1, w2, a0, gew_r, gaw_r, ex_ref, p_ref):
        ne = jnp.dot(e_ref[...].astype(jnp.bfloat16), w0[...],
                     preferred_element_type=jnp.float32)
        ne += jnp.dot(s_ref[...].astype(jnp.bfloat16), w1[...],
                      preferred_element_type=jnp.float32)
        ne += jnp.dot(r_ref[...].astype(jnp.bfloat16), w2[...],
                      preferred_element_type=jnp.float32)
        ne = jnp.maximum(ne + gew_r[...], 0.0)
        lg = jnp.dot(ne.astype(jnp.bfloat16), a0[...],
                     preferred_element_type=jnp.float32)
        lg = jnp.maximum(lg + gaw_r[...], 0.0)
        ex = jnp.exp(lg)
        ex_ref[...] = ex
        p_ref[...] = ex * ne

    blk = pl.BlockSpec((brows, D), lambda i: (i, 0))
    wblk = pl.BlockSpec((D, D), lambda i: (0, 0))
    vblk = pl.BlockSpec((1, D), lambda i: (0, 0))
    out = jax.ShapeDtypeStruct((M, D), jnp.float32)
    bf = jnp.bfloat16
    return pl.pallas_call(
        body,
        grid=(M // brows,),
        in_specs=[blk, blk, blk, wblk, wblk, wblk, wblk, vblk, vblk],
        out_specs=(blk, blk),
        out_shape=(out, out),
    )(elat, sent, recv, W0.astype(bf), W1.astype(bf), W2.astype(bf),
      A0.astype(bf), gew, gaw)


def _nd_combine_tc(dq, brows=2048):
    N = dq.shape[1]

    def body(a_ref, ri_ref):
        d = a_ref[0] + a_ref[1]
        ri_ref[...] = 1.0 / jnp.where(d == 0.0, 1.0, d)

    blk3 = pl.BlockSpec((2, brows, D), lambda i: (0, i, 0))
    blk = pl.BlockSpec((brows, D), lambda i: (i, 0))
    return pl.pallas_call(
        body,
        grid=(N // brows,),
        in_specs=[blk3],
        out_specs=blk,
        out_shape=jax.ShapeDtypeStruct((N, D), jnp.float32),
    )(dq)


def _att_fused_tc(p, rinvr, elat, ln_scale, ln_bias, brows=4000):
    M = p.shape[0]

    def body(p_ref, dr_ref, el_ref, sc_ref, bi_ref, att_ref, eo_ref, ea_ref):
        att = p_ref[...] * dr_ref[...]
        att_ref[...] = att
        x = att + el_ref[...]
        mu = jnp.mean(x, axis=-1, keepdims=True)
        xc = x - mu
        var = jnp.mean(xc * xc, axis=-1, keepdims=True)
        eo_ref[...] = xc * lax.rsqrt(var + LN_EPS) * sc_ref[...] + bi_ref[...]

        @pl.when(pl.program_id(0) == 0)
        def _():
            ea_ref[...] = jnp.zeros_like(ea_ref)

        ea_ref[...] += jnp.sum(att, axis=0, keepdims=True)

    blk = pl.BlockSpec((brows, D), lambda i: (i, 0))
    vblk = pl.BlockSpec((1, D), lambda i: (0, 0))
    out = jax.ShapeDtypeStruct((M, D), jnp.float32)
    outv = jax.ShapeDtypeStruct((1, D), jnp.float32)
    return pl.pallas_call(
        body,
        grid=(M // brows,),
        in_specs=[blk, blk, blk, vblk, vblk],
        out_specs=(blk, blk, vblk),
        out_shape=(out, out, outv),
    )(p, rinvr, elat, ln_scale.reshape(1, D), ln_bias.reshape(1, D))


def _node_fused_tc(nlat, spa, spb, Wn0, Wn1, Wn2, gnw, ln_scale,
                   ln_bias, brows=2000):
    N = nlat.shape[0]

    def body(n_ref, spa_ref, spb_ref, w0, w1, w2, gnw_r, sc_ref, bi_ref,
             no_ref, na_ref):
        sa = spa_ref[0] + spb_ref[0]
        rv = spa_ref[1] + spb_ref[1]
        nn = jnp.dot(n_ref[...], w0[...], preferred_element_type=jnp.float32)
        nn += jnp.dot(sa, w1[...], preferred_element_type=jnp.float32)
        nn += jnp.dot(rv, w2[...], preferred_element_type=jnp.float32)
        nn = jnp.maximum(nn + gnw_r[...], 0.0)

        @pl.when(pl.program_id(0) == 0)
        def _():
            na_ref[...] = jnp.zeros_like(na_ref)

        na_ref[...] += jnp.sum(nn, axis=0, keepdims=True)
        x = nn + n_ref[...]
        mu = jnp.mean(x, axis=-1, keepdims=True)
        xc = x - mu
        var = jnp.mean(xc * xc, axis=-1, keepdims=True)
        no_ref[...] = xc * lax.rsqrt(var + LN_EPS) * sc_ref[...] + bi_ref[...]

    blk = pl.BlockSpec((brows, D), lambda i: (i, 0))
    blk3 = pl.BlockSpec((2, brows, D), lambda i: (0, i, 0))
    wblk = pl.BlockSpec((D, D), lambda i: (0, 0))
    vblk = pl.BlockSpec((1, D), lambda i: (0, 0))
    return pl.pallas_call(
        body,
        grid=(N // brows,),
        in_specs=[blk, blk3, blk3, wblk, wblk, wblk, vblk, vblk, vblk],
        out_specs=(blk, vblk),
        out_shape=(jax.ShapeDtypeStruct((N, D), jnp.float32),
                   jax.ShapeDtypeStruct((1, D), jnp.float32)),
    )(nlat, spa, spb, Wn0, Wn1, Wn2, gnw,
      ln_scale.reshape(1, D), ln_bias.reshape(1, D))


def _gup_tc(node_attr, ea0, ea1, g, G0, G1, G2, bg, ln_scale, ln_bias):
    def body(na_ref, ea0_ref, ea1_ref, g_ref, g0, g1, g2, bg_r, sc_ref,
             bi_ref, go_ref):
        ng = jnp.dot(na_ref[...], g0[...], preferred_element_type=jnp.float32)
        ng += jnp.dot(ea0_ref[...] + ea1_ref[...], g1[...],
                      preferred_element_type=jnp.float32)
        ng += jnp.dot(g_ref[...], g2[...], preferred_element_type=jnp.float32)
        ng = jnp.maximum(ng + bg_r[...], 0.0)
        x = ng + g_ref[...]
        mu = jnp.mean(x, axis=-1, keepdims=True)
        xc = x - mu
        var = jnp.mean(xc * xc, axis=-1, keepdims=True)
        go_ref[...] = xc * lax.rsqrt(var + LN_EPS) * sc_ref[...] + bi_ref[...]

    return pl.pallas_call(
        body, out_shape=jax.ShapeDtypeStruct((1, D), jnp.float32))(
            node_attr, ea0, ea1, g, G0, G1, G2, bg.reshape(1, D),
            ln_scale.reshape(1, D), ln_bias.reshape(1, D))


def _concat_rows_tc(a, b, W, bias, brows=2000):
    M = a.shape[0]
    K = a.shape[1]
    Dout = W.shape[1]
    nb = M // brows

    def body(a_ref, b_ref, w_ref, bias_ref, o_ref):
        i = pl.program_id(0)
        x = jnp.where(i < nb, a_ref[...], b_ref[...])
        o_ref[...] = jnp.dot(x, w_ref[...],
                             preferred_element_type=jnp.float32) + bias_ref[...]

    def amap(i):
        return (jnp.minimum(i, nb - 1), 0)

    def bmap(i):
        return (jnp.maximum(i - nb, 0), 0)

    return pl.pallas_call(
        body,
        grid=(2 * nb,),
        in_specs=[
            pl.BlockSpec((brows, K), amap),
            pl.BlockSpec((brows, K), bmap),
            pl.BlockSpec((K, Dout), lambda i: (0, 0)),
            pl.BlockSpec((1, Dout), lambda i: (0, 0)),
        ],
        out_specs=pl.BlockSpec((brows, Dout), lambda i: (i, 0)),
        out_shape=jax.ShapeDtypeStruct((2 * M, Dout), jnp.float32),
    )(a, b, W, bias.reshape(1, Dout))


_MESH = plsc.VectorSubcoreMesh(core_axis_name="c", subcore_axis_name="s",
                               num_cores=NC, num_subcores=NS)

_G_CHUNK = 128
_G_NCH = EH // _G_CHUNK
_G_PW = _G_NCH // NW
_G_EXTRA = _G_NCH - _G_PW * NW
_G_RING = 3


def _sc_gather_one(table_hbm, idx_hbm, out_hbm, ibufs, bufs, isems, gsems,
                   osems, wid):
    R = _G_RING
    F3 = (_G_PW // R) * R
    rem = _G_PW - F3

    def fire_i(j, b):
        c = (wid + j * NW) * _G_CHUNK
        pltpu.async_copy(idx_hbm.at[pl.ds(c, _G_CHUNK)], ibufs[b], isems[b])

    def wait_i(b):
        pltpu.make_async_copy(idx_hbm.at[pl.ds(0, _G_CHUNK)], ibufs[b],
                              isems[b]).wait()

    def fire_g(b):
        pltpu.async_copy(table_hbm.at[ibufs[b]], bufs[b], gsems[b])

    def wait_g(b):
        pltpu.make_async_copy(out_hbm.at[pl.ds(0, _G_CHUNK)], bufs[b],
                              gsems[b]).wait()

    def fire_o(j, b):
        c = (wid + j * NW) * _G_CHUNK
        pltpu.async_copy(bufs[b], out_hbm.at[pl.ds(c, _G_CHUNK)], osems[b])

    def wait_o(b):
        pltpu.make_async_copy(out_hbm.at[pl.ds(0, _G_CHUNK)], bufs[b],
                              osems[b]).wait()

    for b in range(R):
        fire_i(b, b)
    for b in range(R):
        wait_i(b)
        fire_g(b)

    def body(k, carry):
        for b in range(R):
            wait_g(b)
            fire_o(k * R + b, b)
        for b in range(R):
            wait_o(b)
            fire_i(k * R + b + R, b)
        for b in range(R):
            wait_i(b)
            fire_g(b)
        return carry

    lax.fori_loop(0, F3 // R - 1, body, 0)
    for b in range(R):
        wait_g(b)
        fire_o(F3 - R + b, b)
    for b in range(R):
        wait_o(b)
    for j in range(rem):
        fire_i(F3 + j, 0)
        wait_i(0)
        fire_g(0)
        wait_g(0)
        fire_o(F3 + j, 0)
        wait_o(0)

    @pl.when(wid < _G_EXTRA)
    def _():
        fire_i(_G_PW, 0)
        wait_i(0)
        fire_g(0)
        wait_g(0)
        fire_o(_G_PW, 0)
        wait_o(0)


_G_SCRATCH = [
    pltpu.VMEM((_G_CHUNK,), jnp.int32),
    pltpu.VMEM((_G_CHUNK,), jnp.int32),
    pltpu.VMEM((_G_CHUNK,), jnp.int32),
    pltpu.VMEM((_G_CHUNK, D), jnp.float32),
    pltpu.VMEM((_G_CHUNK, D), jnp.float32),
    pltpu.VMEM((_G_CHUNK, D), jnp.float32),
] + [pltpu.SemaphoreType.DMA] * 9 + [
    pltpu.VMEM_SHARED((N_NODES, D), jnp.float32),
]

_T_ROWS = 632


def _stage_table(table_hbm, tbl, sid):
    @pl.when(sid < NS - 1)
    def _():
        pltpu.sync_copy(table_hbm.at[pl.ds(sid * _T_ROWS, _T_ROWS)],
                        tbl.at[pl.ds(sid * _T_ROWS, _T_ROWS)])

    @pl.when(sid == NS - 1)
    def _():
        last = N_NODES - (NS - 1) * _T_ROWS
        pltpu.sync_copy(table_hbm.at[pl.ds((NS - 1) * _T_ROWS, last)],
                        tbl.at[pl.ds((NS - 1) * _T_ROWS, last)])

    plsc.subcore_barrier()


def _sc_gather2(table, senders, receivers):
    @functools.partial(
        pl.kernel,
        out_type=(jax.ShapeDtypeStruct((EH, D), jnp.float32),
                  jax.ShapeDtypeStruct((EH, D), jnp.float32)),
        mesh=_MESH,
        scratch_types=_G_SCRATCH,
    )
    def k(table_hbm, s_hbm, r_hbm, os_hbm, or_hbm, i0, i1, i2, b0, b1, b2,
          s0, s1, s2, g0, g1, g2, o0, o1, o2, tbl):
        cid = lax.axis_index("c")
        sid = lax.axis_index("s")
        wid = sid * NC + cid
        _stage_table(table_hbm, tbl, sid)
        _sc_gather_one(tbl, s_hbm, os_hbm, (i0, i1, i2), (b0, b1, b2),
                       (s0, s1, s2), (g0, g1, g2), (o0, o1, o2), wid)
        _sc_gather_one(tbl, r_hbm, or_hbm, (i0, i1, i2), (b0, b1, b2),
                       (s0, s1, s2), (g0, g1, g2), (o0, o1, o2), wid)

    return k(table, senders, receivers)


def _sc_gather1(table, idx):
    @functools.partial(
        pl.kernel,
        out_type=jax.ShapeDtypeStruct((EH, D), jnp.float32),
        mesh=_MESH,
        scratch_types=_G_SCRATCH,
    )
    def k(table_hbm, i_hbm, out_hbm, i0, i1, i2, b0, b1, b2, s0, s1, s2, g0,
          g1, g2, o0, o1, o2, tbl):
        cid = lax.axis_index("c")
        sid = lax.axis_index("s")
        wid = sid * NC + cid
        _stage_table(table_hbm, tbl, sid)
        _sc_gather_one(tbl, i_hbm, out_hbm, (i0, i1, i2), (b0, b1, b2),
                       (s0, s1, s2), (g0, g1, g2), (o0, o1, o2), wid)

    return k(table, idx)


_S_CHUNK = 40
N_PAD = 10240
_Z_ROWS = N_PAD // NS
_S_RING = 5
_S_SEC = 5
_S_NCH = (EH // NS) // _S_CHUNK
_S_SECN = _S_NCH // _S_SEC


def _sc_scatter_loop(val_hbm, base, slab_fn, idx_slab, vbufs, vsems, ssems,
                     acc):
    R = _S_RING

    def fire_v(c, b):
        pltpu.async_copy(val_hbm.at[pl.ds(base + c * _S_CHUNK, _S_CHUNK)],
                         vbufs[b], vsems[b])

    def wait_v(b):
        pltpu.make_async_copy(val_hbm.at[pl.ds(base, _S_CHUNK)], vbufs[b],
                              vsems[b]).wait()

    def fire_s(j, b):
        pltpu.async_copy(vbufs[b], acc.at[idx_slab.at[j]], ssems[b], add=True)

    def wait_s(b):
        pltpu.make_async_copy(val_hbm.at[pl.ds(base, _S_CHUNK)], vbufs[b],
                              ssems[b]).wait()

    def section(s, carry):
        pltpu.sync_copy(slab_fn(s), idx_slab)
        c0 = s * _S_SECN
        for b in range(R):
            fire_v(c0 + b, b)

        def body(k, carry2):
            for b in range(R):
                wait_v(b)
                fire_s(k * R + b, b)
            for b in range(R):
                wait_s(b)
                fire_v(c0 + k * R + b + R, b)
            return carry2

        lax.fori_loop(0, _S_SECN // R - 1, body, 0)
        for b in range(R):
            wait_v(b)
            fire_s(_S_SECN - R + b, b)
        for b in range(R):
            wait_s(b)
        return carry

    lax.fori_loop(0, _S_SEC, section, 0)


_S_SCRATCH = [
    pltpu.VMEM((_S_SECN, _S_CHUNK), jnp.int32),
    pltpu.VMEM((_S_CHUNK, D), jnp.float32),
    pltpu.VMEM((_S_CHUNK, D), jnp.float32),
    pltpu.VMEM((_S_CHUNK, D), jnp.float32),
    pltpu.VMEM((_S_CHUNK, D), jnp.float32),
    pltpu.VMEM((_S_CHUNK, D), jnp.float32),
] + [pltpu.SemaphoreType.DMA] * 10 + [
    pltpu.VMEM_SHARED((N_PAD, D), jnp.float32),
]


def _sc_segsum_ex(ex0, ex1, ridx6, zeros):
    @functools.partial(
        pl.kernel,
        out_type=jax.ShapeDtypeStruct((NC * N_PAD, D), jnp.float32),
        mesh=_MESH,
        scratch_types=_S_SCRATCH,
    )
    def k(e0_hbm, e1_hbm, idx6_hbm, z_hbm, out_hbm, idx_slab, v0, v1, v2, v3,
          v4, s0, s1, s2, s3, s4, t0, t1, t2, t3, t4, acc):
        cid = lax.axis_index("c")
        sid = lax.axis_index("s")
        pltpu.sync_copy(z_hbm, acc.at[pl.ds(sid * _Z_ROWS, _Z_ROWS)])
        plsc.subcore_barrier()
        vbufs = (v0, v1, v2, v3, v4)
        vsems = (s0, s1, s2, s3, s4)
        ssems = (t0, t1, t2, t3, t4)
        base = sid * (EH // NS)

        @pl.when(cid == 0)
        def _():
            _sc_scatter_loop(e0_hbm, base, lambda s: idx6_hbm.at[0, sid, s],
                             idx_slab, vbufs, vsems, ssems, acc)

        @pl.when(cid == 1)
        def _():
            _sc_scatter_loop(e1_hbm, base, lambda s: idx6_hbm.at[1, sid, s],
                             idx_slab, vbufs, vsems, ssems, acc)

        plsc.subcore_barrier()
        pltpu.sync_copy(acc.at[pl.ds(sid * _Z_ROWS, _Z_ROWS)],
                        out_hbm.at[pl.ds(cid * N_PAD + sid * _Z_ROWS,
                                         _Z_ROWS)])

    return k(ex0, ex1, ridx6, zeros)


def _sc_segsum_sr(att, sidx5, ridx5, zeros):
    @functools.partial(
        pl.kernel,
        out_type=jax.ShapeDtypeStruct((NC * N_PAD, D), jnp.float32),
        mesh=_MESH,
        scratch_types=_S_SCRATCH,
    )
    def k(a_hbm, sidx_hbm, ridx_hbm, z_hbm, out_hbm, idx_slab, v0, v1, v2,
          v3, v4, s0, s1, s2, s3, s4, t0, t1, t2, t3, t4, acc):
        cid = lax.axis_index("c")
        sid = lax.axis_index("s")
        pltpu.sync_copy(z_hbm, acc.at[pl.ds(sid * _Z_ROWS, _Z_ROWS)])
        plsc.subcore_barrier()
        vbufs = (v0, v1, v2, v3, v4)
        vsems = (s0, s1, s2, s3, s4)
        ssems = (t0, t1, t2, t3, t4)
        base = sid * (EH // NS)

        @pl.when(cid == 0)
        def _():
            _sc_scatter_loop(a_hbm, base, lambda s: sidx_hbm.at[sid, s],
                             idx_slab, vbufs, vsems, ssems, acc)

        @pl.when(cid == 1)
        def _():
            _sc_scatter_loop(a_hbm, base, lambda s: ridx_hbm.at[sid, s],
                             idx_slab, vbufs, vsems, ssems, acc)

        plsc.subcore_barrier()
        pltpu.sync_copy(acc.at[pl.ds(sid * _Z_ROWS, _Z_ROWS)],
                        out_hbm.at[pl.ds(cid * N_PAD + sid * _Z_ROWS,
                                         _Z_ROWS)])

    return k(att, sidx5, ridx5, zeros)


def kernel(nodes, edges, globals_, senders, receivers, params):
    zeros = jnp.zeros((_Z_ROWS, D), jnp.float32)
    s_h = (senders[:EH], senders[EH:])
    r_h = (receivers[:EH], receivers[EH:])
    ridx6 = receivers.reshape(2, NS, _S_SEC, _S_SECN, _S_CHUNK)
    sidx6 = senders.reshape(2, NS, _S_SEC, _S_SECN, _S_CHUNK)

    nlat = _dense_tc(nodes, params["embed_node"]["W"],
                     params["embed_node"]["b"], brows=2000)
    elat = [
        _dense_tc(edges, params["embed_edge"]["W"], params["embed_edge"]["b"],
                  brows=2000, in_off=h * (EH // 2000), rows=EH)
        for h in range(2)
    ]
    g = _dense_tc(globals_, params["embed_global"]["W"],
                  params["embed_global"]["b"], brows=1)

    for s in range(2):
        sp = params["step%d" % s]
        We, be = sp["edge_mlp"][0]["W"], sp["edge_mlp"][0]["b"]
        Wa, ba = sp["attn_mlp"][0]["W"], sp["attn_mlp"][0]["b"]
        Wn, bn = sp["node_mlp"][0]["W"], sp["node_mlp"][0]["b"]
        Wg, bg = sp["global_mlp"][0]["W"], sp["global_mlp"][0]["b"]

        gew, gaw, gnw = _prep_tc(g, We[384:512], be, Wa[128:256], ba,
                                 Wn[384:512], bn)
        sr = [_sc_gather2(nlat, s_h[h], r_h[h]) for h in range(2)]
        ex0, p0 = _edge_fused_tc(elat[0], sr[0][0], sr[0][1], We[0:128],
                                 We[128:256], We[256:384], Wa[0:128], gew,
                                 gaw)
        ex1, p1 = _edge_fused_tc(elat[1], sr[1][0], sr[1][1], We[0:128],
                                 We[128:256], We[256:384], Wa[0:128], gew,
                                 gaw)
        dq = _sc_segsum_ex(ex0, ex1, ridx6, zeros).reshape(NC, N_PAD, D)
        rinv = _nd_combine_tc(dq)
        rr0 = _sc_gather1(rinv, r_h[0])
        rr1 = _sc_gather1(rinv, r_h[1])
        att0, el0, ea0 = _att_fused_tc(p0, rr0, elat[0],
                                       sp["ln_edges"]["scale"],
                                       sp["ln_edges"]["bias"])
        spa = _sc_segsum_sr(att0, sidx6[0], ridx6[0],
                            zeros).reshape(NC, N_PAD, D)
        att1, el1, ea1 = _att_fused_tc(p1, rr1, elat[1],
                                       sp["ln_edges"]["scale"],
                                       sp["ln_edges"]["bias"])
        spb = _sc_segsum_sr(att1, sidx6[1], ridx6[1],
                            zeros).reshape(NC, N_PAD, D)
        elat = [el0, el1]
        aee = ((att0, el0, ea0), (att1, el1, ea1))
        nlat, node_attr = _node_fused_tc(
            nlat, spa, spb, Wn[0:128], Wn[128:256], Wn[256:384], gnw,
            sp["ln_nodes"]["scale"], sp["ln_nodes"]["bias"])
        g = _gup_tc(node_attr, aee[0][2], aee[1][2], g, Wg[0:128],
                    Wg[128:256], Wg[256:384], bg, sp["ln_globals"]["scale"],
                    sp["ln_globals"]["bias"])

    nodes_o = _dense_tc(nlat, params["decode_node"]["W"],
                        params["decode_node"]["b"], brows=2000)
    edges_o = _concat_rows_tc(elat[0], elat[1], params["decode_edge"]["W"],
                              params["decode_edge"]["b"], brows=2000)
    g_o = _dense_tc(g, params["decode_global"]["W"],
                    params["decode_global"]["b"], brows=1)
    return nodes_o, edges_o, g_o


# speedup vs baseline: 1.1470x; 1.0090x over previous
import functools

import jax
import jax.numpy as jnp
from jax import lax
from jax.experimental import pallas as pl
from jax.experimental.pallas import tpu as pltpu
from jax.experimental.pallas import tpu_sc as plsc

N_NODES = 10000
N_EDGES = 160000
EH = N_EDGES // 2
D = 128
LN_EPS = 1e-6

NC = 2
NS = 16
NW = NC * NS


def _dense_tc(x, W, b, brows, in_off=0, rows=None):
    M, K = x.shape
    rows = rows if rows is not None else M
    Dout = W.shape[1]
    b2 = b.reshape(1, Dout)

    def body(x_ref, w_ref, b_ref, o_ref):
        y = jnp.dot(x_ref[...], w_ref[...],
                    preferred_element_type=jnp.float32) + b_ref[...]
        o_ref[...] = y

    return pl.pallas_call(
        body,
        grid=(rows // brows,),
        in_specs=[
            pl.BlockSpec((brows, K), lambda i: (i + in_off, 0)),
            pl.BlockSpec((K, Dout), lambda i: (0, 0)),
            pl.BlockSpec((1, Dout), lambda i: (0, 0)),
        ],
        out_specs=pl.BlockSpec((brows, Dout), lambda i: (i, 0)),
        out_shape=jax.ShapeDtypeStruct((rows, Dout), jnp.float32),
    )(x, W, b2)


def _prep_tc(g, Weg, be, Wag, ba, Wng, bn):
    def body(g_ref, weg, be_r, wag, ba_r, wng, bn_r, gew, gaw, gnw):
        gv = g_ref[...]
        gew[...] = jnp.dot(gv, weg[...], preferred_element_type=jnp.float32) + be_r[...]
        gaw[...] = jnp.dot(gv, wag[...], preferred_element_type=jnp.float32) + ba_r[...]
        gnw[...] = jnp.dot(gv, wng[...], preferred_element_type=jnp.float32) + bn_r[...]

    out = jax.ShapeDtypeStruct((1, D), jnp.float32)
    return pl.pallas_call(body, out_shape=(out, out, out))(
        g, Weg, be.reshape(1, D), Wag, ba.reshape(1, D), Wng, bn.reshape(1, D))


def _edge_fused_tc(elat, sent, recv, W0, W1, W2, A0, gew, gaw, brows=8000):
    M = elat.shape[0]

    def body(e_ref, s_ref, r_ref, w0, w1, w2, a0, gew_r, gaw_r, ex_ref, p_ref):
        ne = jnp.dot(e_ref[...].astype(jnp.bfloat16), w0[...],
                     preferred_element_type=jnp.float32)
        ne += jnp.dot(s_ref[...].astype(jnp.bfloat16), w1[...],
                      preferred_element_type=jnp.float32)
        ne += jnp.dot(r_ref[...].astype(jnp.bfloat16), w2[...],
                      preferred_element_type=jnp.float32)
        ne = jnp.maximum(ne + gew_r[...], 0.0)
        lg = jnp.dot(ne.astype(jnp.bfloat16), a0[...],
                     preferred_element_type=jnp.float32)
        lg = jnp.maximum(lg + gaw_r[...], 0.0)
        ex = jnp.exp(lg)
        ex_ref[...] = ex
        p_ref[...] = ex * ne

    blk = pl.BlockSpec((brows, D), lambda i: (i, 0))
    wblk = pl.BlockSpec((D, D), lambda i: (0, 0))
    vblk = pl.BlockSpec((1, D), lambda i: (0, 0))
    out = jax.ShapeDtypeStruct((M, D), jnp.float32)
    bf = jnp.bfloat16
    return pl.pallas_call(
        body,
        grid=(M // brows,),
        in_specs=[blk, blk, blk, wblk, wblk, wblk, wblk, vblk, vblk],
        out_specs=(blk, blk),
        out_shape=(out, out),
    )(elat, sent, recv, W0.astype(bf), W1.astype(bf), W2.astype(bf),
      A0.astype(bf), gew, gaw)


def _nd_combine_tc(dq, brows=2048):
    N = dq.shape[1]

    def body(a_ref, ri_ref):
        d = a_ref[0] + a_ref[1]
        ri_ref[...] = 1.0 / jnp.where(d == 0.0, 1.0, d)

    blk3 = pl.BlockSpec((2, brows, D), lambda i: (0, i, 0))
    blk = pl.BlockSpec((brows, D), lambda i: (i, 0))
    return pl.pallas_call(
        body,
        grid=(N // brows,),
        in_specs=[blk3],
        out_specs=blk,
        out_shape=jax.ShapeDtypeStruct((N, D), jnp.float32),
    )(dq)


def _att_fused_tc(p, rinvr, elat, ln_scale, ln_bias, brows=8000):
    M = p.shape[0]

    def body(p_ref, dr_ref, el_ref, sc_ref, bi_ref, att_ref, eo_ref, ea_ref):
        att = p_ref[...] * dr_ref[...]
        att_ref[...] = att
        x = att + el_ref[...]
        mu = jnp.mean(x, axis=-1, keepdims=True)
        xc = x - mu
        var = jnp.mean(xc * xc, axis=-1, keepdims=True)
        eo_ref[...] = xc * lax.rsqrt(var + LN_EPS) * sc_ref[...] + bi_ref[...]

        @pl.when(pl.program_id(0) == 0)
        def _():
            ea_ref[...] = jnp.zeros_like(ea_ref)

        ea_ref[...] += jnp.sum(att, axis=0, keepdims=True)

    blk = pl.BlockSpec((brows, D), lambda i: (i, 0))
    vblk = pl.BlockSpec((1, D), lambda i: (0, 0))
    out = jax.ShapeDtypeStruct((M, D), jnp.float32)
    outv = jax.ShapeDtypeStruct((1, D), jnp.float32)
    return pl.pallas_call(
        body,
        grid=(M // brows,),
        in_specs=[blk, blk, blk, vblk, vblk],
        out_specs=(blk, blk, vblk),
        out_shape=(out, out, outv),
    )(p, rinvr, elat, ln_scale.reshape(1, D), ln_bias.reshape(1, D))


def _node_fused_tc(nlat, spa, spb, Wn0, Wn1, Wn2, gnw, ln_scale,
                   ln_bias, brows=5000):
    N = nlat.shape[0]

    def body(n_ref, spa_ref, spb_ref, w0, w1, w2, gnw_r, sc_ref, bi_ref,
             no_ref, na_ref):
        sa = spa_ref[0] + spb_ref[0]
        rv = spa_ref[1] + spb_ref[1]
        nn = jnp.dot(n_ref[...], w0[...], preferred_element_type=jnp.float32)
        nn += jnp.dot(sa, w1[...], preferred_element_type=jnp.float32)
        nn += jnp.dot(rv, w2[...], preferred_element_type=jnp.float32)
        nn = jnp.maximum(nn + gnw_r[...], 0.0)

        @pl.when(pl.program_id(0) == 0)
        def _():
            na_ref[...] = jnp.zeros_like(na_ref)

        na_ref[...] += jnp.sum(nn, axis=0, keepdims=True)
        x = nn + n_ref[...]
        mu = jnp.mean(x, axis=-1, keepdims=True)
        xc = x - mu
        var = jnp.mean(xc * xc, axis=-1, keepdims=True)
        no_ref[...] = xc * lax.rsqrt(var + LN_EPS) * sc_ref[...] + bi_ref[...]

    blk = pl.BlockSpec((brows, D), lambda i: (i, 0))
    blk3 = pl.BlockSpec((2, brows, D), lambda i: (0, i, 0))
    wblk = pl.BlockSpec((D, D), lambda i: (0, 0))
    vblk = pl.BlockSpec((1, D), lambda i: (0, 0))
    return pl.pallas_call(
        body,
        grid=(N // brows,),
        in_specs=[blk, blk3, blk3, wblk, wblk, wblk, vblk, vblk, vblk],
        out_specs=(blk, vblk),
        out_shape=(jax.ShapeDtypeStruct((N, D), jnp.float32),
                   jax.ShapeDtypeStruct((1, D), jnp.float32)),
    )(nlat, spa, spb, Wn0, Wn1, Wn2, gnw,
      ln_scale.reshape(1, D), ln_bias.reshape(1, D))


def _gup_tc(node_attr, ea0, ea1, g, G0, G1, G2, bg, ln_scale, ln_bias):
    def body(na_ref, ea0_ref, ea1_ref, g_ref, g0, g1, g2, bg_r, sc_ref,
             bi_ref, go_ref):
        ng = jnp.dot(na_ref[...], g0[...], preferred_element_type=jnp.float32)
        ng += jnp.dot(ea0_ref[...] + ea1_ref[...], g1[...],
                      preferred_element_type=jnp.float32)
        ng += jnp.dot(g_ref[...], g2[...], preferred_element_type=jnp.float32)
        ng = jnp.maximum(ng + bg_r[...], 0.0)
        x = ng + g_ref[...]
        mu = jnp.mean(x, axis=-1, keepdims=True)
        xc = x - mu
        var = jnp.mean(xc * xc, axis=-1, keepdims=True)
        go_ref[...] = xc * lax.rsqrt(var + LN_EPS) * sc_ref[...] + bi_ref[...]

    return pl.pallas_call(
        body, out_shape=jax.ShapeDtypeStruct((1, D), jnp.float32))(
            node_attr, ea0, ea1, g, G0, G1, G2, bg.reshape(1, D),
            ln_scale.reshape(1, D), ln_bias.reshape(1, D))


def _concat_rows_tc(a, b, W, bias, brows=2000):
    M = a.shape[0]
    K = a.shape[1]
    Dout = W.shape[1]
    nb = M // brows

    def body(a_ref, b_ref, w_ref, bias_ref, o_ref):
        i = pl.program_id(0)
        x = jnp.where(i < nb, a_ref[...], b_ref[...])
        o_ref[...] = jnp.dot(x, w_ref[...],
                             preferred_element_type=jnp.float32) + bias_ref[...]

    def amap(i):
        return (jnp.minimum(i, nb - 1), 0)

    def bmap(i):
        return (jnp.maximum(i - nb, 0), 0)

    return pl.pallas_call(
        body,
        grid=(2 * nb,),
        in_specs=[
            pl.BlockSpec((brows, K), amap),
            pl.BlockSpec((brows, K), bmap),
            pl.BlockSpec((K, Dout), lambda i: (0, 0)),
            pl.BlockSpec((1, Dout), lambda i: (0, 0)),
        ],
        out_specs=pl.BlockSpec((brows, Dout), lambda i: (i, 0)),
        out_shape=jax.ShapeDtypeStruct((2 * M, Dout), jnp.float32),
    )(a, b, W, bias.reshape(1, Dout))


_MESH = plsc.VectorSubcoreMesh(core_axis_name="c", subcore_axis_name="s",
                               num_cores=NC, num_subcores=NS)

_G_CHUNK = 128
_G_NCH = EH // _G_CHUNK
_G_PW = _G_NCH // NW
_G_EXTRA = _G_NCH - _G_PW * NW
_G_RING = 3


def _sc_gather_one(table_hbm, idx_hbm, out_hbm, ibufs, bufs, isems, gsems,
                   osems, wid):
    R = _G_RING
    F3 = (_G_PW // R) * R
    rem = _G_PW - F3

    def fire_i(j, b):
        c = (wid + j * NW) * _G_CHUNK
        pltpu.async_copy(idx_hbm.at[pl.ds(c, _G_CHUNK)], ibufs[b], isems[b])

    def wait_i(b):
        pltpu.make_async_copy(idx_hbm.at[pl.ds(0, _G_CHUNK)], ibufs[b],
                              isems[b]).wait()

    def fire_g(b):
        pltpu.async_copy(table_hbm.at[ibufs[b]], bufs[b], gsems[b])

    def wait_g(b):
        pltpu.make_async_copy(out_hbm.at[pl.ds(0, _G_CHUNK)], bufs[b],
                              gsems[b]).wait()

    def fire_o(j, b):
        c = (wid + j * NW) * _G_CHUNK
        pltpu.async_copy(bufs[b], out_hbm.at[pl.ds(c, _G_CHUNK)], osems[b])

    def wait_o(b):
        pltpu.make_async_copy(out_hbm.at[pl.ds(0, _G_CHUNK)], bufs[b],
                              osems[b]).wait()

    for b in range(R):
        fire_i(b, b)
    for b in range(R):
        wait_i(b)
        fire_g(b)

    def body(k, carry):
        for b in range(R):
            wait_g(b)
            fire_o(k * R + b, b)
        for b in range(R):
            wait_o(b)
            fire_i(k * R + b + R, b)
        for b in range(R):
            wait_i(b)
            fire_g(b)
        return carry

    lax.fori_loop(0, F3 // R - 1, body, 0)
    for b in range(R):
        wait_g(b)
        fire_o(F3 - R + b, b)
    for b in range(R):
        wait_o(b)
    for j in range(rem):
        fire_i(F3 + j, 0)
        wait_i(0)
        fire_g(0)
        wait_g(0)
        fire_o(F3 + j, 0)
        wait_o(0)

    @pl.when(wid < _G_EXTRA)
    def _():
        fire_i(_G_PW, 0)
        wait_i(0)
        fire_g(0)
        wait_g(0)
        fire_o(_G_PW, 0)
        wait_o(0)


_G_SCRATCH = [
    pltpu.VMEM((_G_CHUNK,), jnp.int32),
    pltpu.VMEM((_G_CHUNK,), jnp.int32),
    pltpu.VMEM((_G_CHUNK,), jnp.int32),
    pltpu.VMEM((_G_CHUNK, D), jnp.float32),
    pltpu.VMEM((_G_CHUNK, D), jnp.float32),
    pltpu.VMEM((_G_CHUNK, D), jnp.float32),
] + [pltpu.SemaphoreType.DMA] * 9 + [
    pltpu.VMEM_SHARED((N_NODES, D), jnp.float32),
]

_T_ROWS = 632


def _stage_table(table_hbm, tbl, sid):
    @pl.when(sid < NS - 1)
    def _():
        pltpu.sync_copy(table_hbm.at[pl.ds(sid * _T_ROWS, _T_ROWS)],
                        tbl.at[pl.ds(sid * _T_ROWS, _T_ROWS)])

    @pl.when(sid == NS - 1)
    def _():
        last = N_NODES - (NS - 1) * _T_ROWS
        pltpu.sync_copy(table_hbm.at[pl.ds((NS - 1) * _T_ROWS, last)],
                        tbl.at[pl.ds((NS - 1) * _T_ROWS, last)])

    plsc.subcore_barrier()


def _sc_gather2(table, senders, receivers):
    @functools.partial(
        pl.kernel,
        out_type=(jax.ShapeDtypeStruct((EH, D), jnp.float32),
                  jax.ShapeDtypeStruct((EH, D), jnp.float32)),
        mesh=_MESH,
        scratch_types=_G_SCRATCH,
    )
    def k(table_hbm, s_hbm, r_hbm, os_hbm, or_hbm, i0, i1, i2, b0, b1, b2,
          s0, s1, s2, g0, g1, g2, o0, o1, o2, tbl):
        cid = lax.axis_index("c")
        sid = lax.axis_index("s")
        wid = sid * NC + cid
        _stage_table(table_hbm, tbl, sid)
        _sc_gather_one(tbl, s_hbm, os_hbm, (i0, i1, i2), (b0, b1, b2),
                       (s0, s1, s2), (g0, g1, g2), (o0, o1, o2), wid)
        _sc_gather_one(tbl, r_hbm, or_hbm, (i0, i1, i2), (b0, b1, b2),
                       (s0, s1, s2), (g0, g1, g2), (o0, o1, o2), wid)

    return k(table, senders, receivers)


def _sc_gather1(table, idx):
    @functools.partial(
        pl.kernel,
        out_type=jax.ShapeDtypeStruct((EH, D), jnp.float32),
        mesh=_MESH,
        scratch_types=_G_SCRATCH,
    )
    def k(table_hbm, i_hbm, out_hbm, i0, i1, i2, b0, b1, b2, s0, s1, s2, g0,
          g1, g2, o0, o1, o2, tbl):
        cid = lax.axis_index("c")
        sid = lax.axis_index("s")
        wid = sid * NC + cid
        _stage_table(table_hbm, tbl, sid)
        _sc_gather_one(tbl, i_hbm, out_hbm, (i0, i1, i2), (b0, b1, b2),
                       (s0, s1, s2), (g0, g1, g2), (o0, o1, o2), wid)

    return k(table, idx)


_S_CHUNK = 40
N_PAD = 10240
_Z_ROWS = N_PAD // NS
_S_RING = 5
_S_SEC = 5
_S_NCH = (EH // NS) // _S_CHUNK
_S_SECN = _S_NCH // _S_SEC


def _sc_scatter_loop(val_hbm, base, slab_fn, idx_slab, vbufs, vsems, ssems,
                     acc):
    R = _S_RING

    def fire_v(c, b):
        pltpu.async_copy(val_hbm.at[pl.ds(base + c * _S_CHUNK, _S_CHUNK)],
                         vbufs[b], vsems[b])

    def wait_v(b):
        pltpu.make_async_copy(val_hbm.at[pl.ds(base, _S_CHUNK)], vbufs[b],
                              vsems[b]).wait()

    def fire_s(j, b):
        pltpu.async_copy(vbufs[b], acc.at[idx_slab.at[j]], ssems[b], add=True)

    def wait_s(b):
        pltpu.make_async_copy(val_hbm.at[pl.ds(base, _S_CHUNK)], vbufs[b],
                              ssems[b]).wait()

    def section(s, carry):
        pltpu.sync_copy(slab_fn(s), idx_slab)
        c0 = s * _S_SECN
        for b in range(R):
            fire_v(c0 + b, b)

        def body(k, carry2):
            for b in range(R):
                wait_v(b)
                fire_s(k * R + b, b)
            for b in range(R):
                wait_s(b)
                fire_v(c0 + k * R + b + R, b)
            return carry2

        lax.fori_loop(0, _S_SECN // R - 1, body, 0)
        for b in range(R):
            wait_v(b)
            fire_s(_S_SECN - R + b, b)
        for b in range(R):
            wait_s(b)
        return carry

    lax.fori_loop(0, _S_SEC, section, 0)


_S_SCRATCH = [
    pltpu.VMEM((_S_SECN, _S_CHUNK), jnp.int32),
    pltpu.VMEM((_S_CHUNK, D), jnp.float32),
    pltpu.VMEM((_S_CHUNK, D), jnp.float32),
    pltpu.VMEM((_S_CHUNK, D), jnp.float32),
    pltpu.VMEM((_S_CHUNK, D), jnp.float32),
    pltpu.VMEM((_S_CHUNK, D), jnp.float32),
] + [pltpu.SemaphoreType.DMA] * 10 + [
    pltpu.VMEM_SHARED((N_PAD, D), jnp.float32),
]


def _sc_segsum_ex(ex0, ex1, ridx6, zeros):
    @functools.partial(
        pl.kernel,
        out_type=jax.ShapeDtypeStruct((NC * N_PAD, D), jnp.float32),
        mesh=_MESH,
        scratch_types=_S_SCRATCH,
    )
    def k(e0_hbm, e1_hbm, idx6_hbm, z_hbm, out_hbm, idx_slab, v0, v1, v2, v3,
          v4, s0, s1, s2, s3, s4, t0, t1, t2, t3, t4, acc):
        cid = lax.axis_index("c")
        sid = lax.axis_index("s")
        pltpu.sync_copy(z_hbm, acc.at[pl.ds(sid * _Z_ROWS, _Z_ROWS)])
        plsc.subcore_barrier()
        vbufs = (v0, v1, v2, v3, v4)
        vsems = (s0, s1, s2, s3, s4)
        ssems = (t0, t1, t2, t3, t4)
        base = sid * (EH // NS)

        @pl.when(cid == 0)
        def _():
            _sc_scatter_loop(e0_hbm, base, lambda s: idx6_hbm.at[0, sid, s],
                             idx_slab, vbufs, vsems, ssems, acc)

        @pl.when(cid == 1)
        def _():
            _sc_scatter_loop(e1_hbm, base, lambda s: idx6_hbm.at[1, sid, s],
                             idx_slab, vbufs, vsems, ssems, acc)

        plsc.subcore_barrier()
        pltpu.sync_copy(acc.at[pl.ds(sid * _Z_ROWS, _Z_ROWS)],
                        out_hbm.at[pl.ds(cid * N_PAD + sid * _Z_ROWS,
                                         _Z_ROWS)])

    return k(ex0, ex1, ridx6, zeros)


def _sc_segsum_sr(att, sidx5, ridx5, zeros):
    @functools.partial(
        pl.kernel,
        out_type=jax.ShapeDtypeStruct((NC * N_PAD, D), jnp.float32),
        mesh=_MESH,
        scratch_types=_S_SCRATCH,
    )
    def k(a_hbm, sidx_hbm, ridx_hbm, z_hbm, out_hbm, idx_slab, v0, v1, v2,
          v3, v4, s0, s1, s2, s3, s4, t0, t1, t2, t3, t4, acc):
        cid = lax.axis_index("c")
        sid = lax.axis_index("s")
        pltpu.sync_copy(z_hbm, acc.at[pl.ds(sid * _Z_ROWS, _Z_ROWS)])
        plsc.subcore_barrier()
        vbufs = (v0, v1, v2, v3, v4)
        vsems = (s0, s1, s2, s3, s4)
        ssems = (t0, t1, t2, t3, t4)
        base = sid * (EH // NS)

        @pl.when(cid == 0)
        def _():
            _sc_scatter_loop(a_hbm, base, lambda s: sidx_hbm.at[sid, s],
                             idx_slab, vbufs, vsems, ssems, acc)

        @pl.when(cid == 1)
        def _():
            _sc_scatter_loop(a_hbm, base, lambda s: ridx_hbm.at[sid, s],
                             idx_slab, vbufs, vsems, ssems, acc)

        plsc.subcore_barrier()
        pltpu.sync_copy(acc.at[pl.ds(sid * _Z_ROWS, _Z_ROWS)],
                        out_hbm.at[pl.ds(cid * N_PAD + sid * _Z_ROWS,
                                         _Z_ROWS)])

    return k(att, sidx5, ridx5, zeros)


def kernel(nodes, edges, globals_, senders, receivers, params):
    zeros = jnp.zeros((_Z_ROWS, D), jnp.float32)
    s_h = (senders[:EH], senders[EH:])
    r_h = (receivers[:EH], receivers[EH:])
    ridx6 = receivers.reshape(2, NS, _S_SEC, _S_SECN, _S_CHUNK)
    sidx6 = senders.reshape(2, NS, _S_SEC, _S_SECN, _S_CHUNK)

    nlat = _dense_tc(nodes, params["embed_node"]["W"],
                     params["embed_node"]["b"], brows=2000)
    elat = [
        _dense_tc(edges, params["embed_edge"]["W"], params["embed_edge"]["b"],
                  brows=2000, in_off=h * (EH // 2000), rows=EH)
        for h in range(2)
    ]
    g = _dense_tc(globals_, params["embed_global"]["W"],
                  params["embed_global"]["b"], brows=1)

    for s in range(2):
        sp = params["step%d" % s]
        We, be = sp["edge_mlp"][0]["W"], sp["edge_mlp"][0]["b"]
        Wa, ba = sp["attn_mlp"][0]["W"], sp["attn_mlp"][0]["b"]
        Wn, bn = sp["node_mlp"][0]["W"], sp["node_mlp"][0]["b"]
        Wg, bg = sp["global_mlp"][0]["W"], sp["global_mlp"][0]["b"]

        gew, gaw, gnw = _prep_tc(g, We[384:512], be, Wa[128:256], ba,
                                 Wn[384:512], bn)
        sr = [_sc_gather2(nlat, s_h[h], r_h[h]) for h in range(2)]
        ex0, p0 = _edge_fused_tc(elat[0], sr[0][0], sr[0][1], We[0:128],
                                 We[128:256], We[256:384], Wa[0:128], gew,
                                 gaw)
        ex1, p1 = _edge_fused_tc(elat[1], sr[1][0], sr[1][1], We[0:128],
                                 We[128:256], We[256:384], Wa[0:128], gew,
                                 gaw)
        dq = _sc_segsum_ex(ex0, ex1, ridx6, zeros).reshape(NC, N_PAD, D)
        rinv = _nd_combine_tc(dq)
        rr0 = _sc_gather1(rinv, r_h[0])
        rr1 = _sc_gather1(rinv, r_h[1])
        att0, el0, ea0 = _att_fused_tc(p0, rr0, elat[0],
                                       sp["ln_edges"]["scale"],
                                       sp["ln_edges"]["bias"])
        spa = _sc_segsum_sr(att0, sidx6[0], ridx6[0],
                            zeros).reshape(NC, N_PAD, D)
        att1, el1, ea1 = _att_fused_tc(p1, rr1, elat[1],
                                       sp["ln_edges"]["scale"],
                                       sp["ln_edges"]["bias"])
        spb = _sc_segsum_sr(att1, sidx6[1], ridx6[1],
                            zeros).reshape(NC, N_PAD, D)
        elat = [el0, el1]
        aee = ((att0, el0, ea0), (att1, el1, ea1))
        nlat, node_attr = _node_fused_tc(
            nlat, spa, spb, Wn[0:128], Wn[128:256], Wn[256:384], gnw,
            sp["ln_nodes"]["scale"], sp["ln_nodes"]["bias"])
        g = _gup_tc(node_attr, aee[0][2], aee[1][2], g, Wg[0:128],
                    Wg[128:256], Wg[256:384], bg, sp["ln_globals"]["scale"],
                    sp["ln_globals"]["bias"])

    nodes_o = _dense_tc(nlat, params["decode_node"]["W"],
                        params["decode_node"]["b"], brows=2000)
    edges_o = _concat_rows_tc(elat[0], elat[1], params["decode_edge"]["W"],
                              params["decode_edge"]["b"], brows=2000)
    g_o = _dense_tc(g, params["decode_global"]["W"],
                    params["decode_global"]["b"], brows=1)
    return nodes_o, edges_o, g_o
